# baseline (speedup 1.0000x reference)
import jax
import jax.numpy as jnp
from jax import lax
from jax.experimental import pallas as pl
from jax.experimental.pallas import tpu as pltpu
from jax.experimental.pallas import tpu_sc as plsc

NODE_DIM = 128
EDGE_DIM = 16

NC, NS = 2, 16
NW = NC * NS
CH = 80
SUBS = 5

_EPS = 1e-5

_SC_PARAMS = pltpu.CompilerParams(use_tc_tiling_on_sc=False)


def _sc_mesh():
    return plsc.VectorSubcoreMesh(core_axis_name="c", subcore_axis_name="s",
                                  num_cores=NC, num_subcores=NS)


def _proj_body(x_ref, w1s_ref, w1d_ref, wrs_ref, wrd_ref,
               ps_ref, pd_ref, rs_ref, rd_ref):
    xb = x_ref[...]
    ps_ref[...] = jnp.dot(xb, w1s_ref[...], preferred_element_type=jnp.float32)
    pd_ref[...] = jnp.dot(xb, w1d_ref[...], preferred_element_type=jnp.float32)
    rs_ref[...] = jnp.dot(xb, wrs_ref[...], preferred_element_type=jnp.float32)
    rd_ref[...] = jnp.dot(xb, wrd_ref[...], preferred_element_type=jnp.float32)


def _node_proj(x, w1s, w1d, wrs, wrd):
    n = x.shape[0]
    blk = 1000
    return pl.pallas_call(
        _proj_body,
        grid=(n // blk,),
        in_specs=[
            pl.BlockSpec((blk, NODE_DIM), lambda i: (i, 0)),
            pl.BlockSpec((NODE_DIM, NODE_DIM), lambda i: (0, 0)),
            pl.BlockSpec((NODE_DIM, NODE_DIM), lambda i: (0, 0)),
            pl.BlockSpec((NODE_DIM, EDGE_DIM), lambda i: (0, 0)),
            pl.BlockSpec((NODE_DIM, EDGE_DIM), lambda i: (0, 0)),
        ],
        out_specs=[
            pl.BlockSpec((blk, NODE_DIM), lambda i: (i, 0)),
            pl.BlockSpec((blk, NODE_DIM), lambda i: (i, 0)),
            pl.BlockSpec((blk, EDGE_DIM), lambda i: (i, 0)),
            pl.BlockSpec((blk, EDGE_DIM), lambda i: (i, 0)),
        ],
        out_shape=[
            jax.ShapeDtypeStruct((n, NODE_DIM), jnp.float32),
            jax.ShapeDtypeStruct((n, NODE_DIM), jnp.float32),
            jax.ShapeDtypeStruct((n, EDGE_DIM), jnp.float32),
            jax.ShapeDtypeStruct((n, EDGE_DIM), jnp.float32),
        ],
    )(x, w1s, w1d, wrs, wrd)


def _make_gather_body(width):
    def body(ps_hbm, pd_hbm, src_hbm, dst_hbm, h_hbm,
             idx_s, idx_d, bs0, bd0, bs1, bd1, bw0, bw1,
             sg0, sg1, sw0, sw1):
        per_w = idx_s.shape[0]
        n_chunks = per_w // CH
        c = lax.axis_index("c")
        s = lax.axis_index("s")
        wid = c * NS + s
        base = wid * per_w
        pltpu.sync_copy(src_hbm.at[pl.ds(base, per_w)], idx_s)
        pltpu.sync_copy(dst_hbm.at[pl.ds(base, per_w)], idx_d)
        bs = (bs0, bs1)
        bd = (bd0, bd1)
        bw = (bw0, bw1)
        sg = (sg0, sg1)
        sw = (sw0, sw1)

        def fire(t, b):
            tsl = pl.ds(t * CH, CH)
            pltpu.async_copy(ps_hbm.at[idx_s.at[tsl]], bs[b], sg[b])
            pltpu.async_copy(pd_hbm.at[idx_d.at[tsl]], bd[b], sg[b])

        def process(t, b):
            tsl = pl.ds(t * CH, CH)
            pltpu.make_async_copy(ps_hbm.at[idx_s.at[tsl]], bs[b],
                                  sg[b]).wait()
            pltpu.make_async_copy(pd_hbm.at[idx_d.at[tsl]], bd[b],
                                  sg[b]).wait()

            @pl.when(t >= 2)
            def _():
                pltpu.make_async_copy(bw[b], h_hbm.at[pl.ds(base, CH)],
                                      sw[b]).wait()

            def row(i, c2):
                for j in range(width // 16):
                    sl = pl.ds(j * 16, 16)
                    bw[b][i, sl] = bs[b][i, sl] + bd[b][i, sl]
                return c2

            lax.fori_loop(0, CH, row, 0)
            pltpu.async_copy(bw[b], h_hbm.at[pl.ds(base + t * CH, CH)], sw[b])

            @pl.when(t + 2 < n_chunks)
            def _():
                fire(t + 2, b)

        fire(0, 0)
        fire(1, 1)

        def step(tt, carry):
            for b in range(2):
                t = 2 * tt + b

                @pl.when(t < n_chunks)
                def _():
                    process(t, b)

            return carry

        lax.fori_loop(0, (n_chunks + 1) // 2, step, 0)
        pltpu.make_async_copy(bw0, h_hbm.at[pl.ds(base, CH)], sw0).wait()
        pltpu.make_async_copy(bw1, h_hbm.at[pl.ds(base, CH)], sw1).wait()

    return body


def _sc_gather_add(ps, pd, src_f, dst_f, n_edges, width):
    per_w = n_edges // NW
    params = None if width == NODE_DIM else _SC_PARAMS
    return pl.kernel(
        _make_gather_body(width),
        out_type=jax.ShapeDtypeStruct((n_edges, width), jnp.float32),
        mesh=_sc_mesh(),
        compiler_params=params,
        scratch_types=[
            pltpu.VMEM((per_w,), jnp.int32),
            pltpu.VMEM((per_w,), jnp.int32),
            pltpu.VMEM((CH, width), jnp.float32),
            pltpu.VMEM((CH, width), jnp.float32),
            pltpu.VMEM((CH, width), jnp.float32),
            pltpu.VMEM((CH, width), jnp.float32),
            pltpu.VMEM((CH, width), jnp.float32),
            pltpu.VMEM((CH, width), jnp.float32),
            pltpu.SemaphoreType.DMA,
            pltpu.SemaphoreType.DMA,
            pltpu.SemaphoreType.DMA,
            pltpu.SemaphoreType.DMA,
        ],
    )(ps, pd, src_f, dst_f)


def _edge_body(h_ref, r_ref, ea_ref, w1e_ref, b1_ref, w2_ref, wre_ref,
               bz_ref, gam_ref, bet_ref, out_ref):
    ea = ea_ref[...]
    h = (h_ref[...]
         + jnp.dot(ea, w1e_ref[...], preferred_element_type=jnp.float32)
         + b1_ref[...])
    y = h * (1.0 / (1.0 + jnp.exp(-h)))
    z = (jnp.dot(y, w2_ref[...], preferred_element_type=jnp.float32)
         + r_ref[...]
         + jnp.dot(ea, wre_ref[...], preferred_element_type=jnp.float32)
         + bz_ref[...])
    mu = jnp.mean(z, axis=1, keepdims=True)
    zc = z - mu
    var = jnp.mean(zc * zc, axis=1, keepdims=True)
    out_ref[...] = zc * lax.rsqrt(var + _EPS) * gam_ref[...] + bet_ref[...]


def _edge_mlp_fused(h, r, ea, w1e, b1, w2, wre, bz, gam, bet):
    n_edges = h.shape[0]
    blk = 512
    return pl.pallas_call(
        _edge_body,
        grid=(n_edges // blk,),
        in_specs=[
            pl.BlockSpec((blk, NODE_DIM), lambda i: (i, 0)),
            pl.BlockSpec((blk, EDGE_DIM), lambda i: (i, 0)),
            pl.BlockSpec((blk, EDGE_DIM), lambda i: (i, 0)),
            pl.BlockSpec((EDGE_DIM, NODE_DIM), lambda i: (0, 0)),
            pl.BlockSpec((1, NODE_DIM), lambda i: (0, 0)),
            pl.BlockSpec((NODE_DIM, EDGE_DIM), lambda i: (0, 0)),
            pl.BlockSpec((EDGE_DIM, EDGE_DIM), lambda i: (0, 0)),
            pl.BlockSpec((1, EDGE_DIM), lambda i: (0, 0)),
            pl.BlockSpec((1, EDGE_DIM), lambda i: (0, 0)),
            pl.BlockSpec((1, EDGE_DIM), lambda i: (0, 0)),
        ],
        out_specs=pl.BlockSpec((blk, EDGE_DIM), lambda i: (i, 0)),
        out_shape=jax.ShapeDtypeStruct((n_edges, EDGE_DIM), jnp.float32),
    )(h, r, ea, w1e, b1, w2, wre, bz, gam, bet)


def _scatter_body(ne_hbm, dst_hbm, out_hbm, agg_sh, idx_v,
                  rows0, rows1, rows2, zb, sl0, sl1, sl2, ss0, ss1, ss2):
    per_w = idx_v.shape[0]
    n_chunks = per_w // CH
    n_groups = n_chunks // SUBS
    grp = SUBS * CH
    n_nodes = agg_sh.shape[0]
    zrows = zb.shape[0]
    n_zcopies = n_nodes // zrows
    c = lax.axis_index("c")
    s = lax.axis_index("s")
    wid = c * NS + s
    base = wid * per_w
    rows = (rows0, rows1, rows2)
    sl = (sl0, sl1, sl2)
    ss = (ss0, ss1, ss2)

    def zr(i, carry):
        zb[i, :] = jnp.zeros((16,), jnp.float32)
        return carry

    lax.fori_loop(0, zrows, zr, 0)

    def zcopy(j, carry):
        k = s + NS * j

        @pl.when(k < n_zcopies)
        def _():
            pltpu.sync_copy(zb, agg_sh.at[pl.ds(k * zrows, zrows)])

        return carry

    lax.fori_loop(0, (n_zcopies + NS - 1) // NS, zcopy, 0)
    pltpu.sync_copy(dst_hbm.at[pl.ds(base, per_w)], idx_v)
    plsc.subcore_barrier()

    def fire(g, b):
        pltpu.async_copy(ne_hbm.at[pl.ds(base + g * grp, grp)], rows[b], sl[b])

    def drain_scatters(b):
        for j in range(SUBS):
            pltpu.make_async_copy(rows[b].at[pl.ds(j * CH, CH)],
                                  agg_sh.at[idx_v.at[pl.ds(0, CH)]],
                                  ss[b]).wait()

    def process(g, b, bn):
        pltpu.make_async_copy(ne_hbm.at[pl.ds(base, grp)], rows[b],
                              sl[b]).wait()
        for j in range(SUBS):
            isl = pl.ds(g * grp + j * CH, CH)
            pltpu.async_copy(rows[b].at[pl.ds(j * CH, CH)],
                             agg_sh.at[idx_v.at[isl]], ss[b], add=True)

        @pl.when(g >= 1)
        def _():
            drain_scatters(bn)

        @pl.when(g + 2 < n_groups)
        def _():
            fire(g + 2, bn)

    fire(0, 0)
    fire(1, 1)

    def step(gg, carry):
        for b in range(3):
            g = 3 * gg + b

            @pl.when(g < n_groups)
            def _():
                process(g, b, (b + 2) % 3)

        return carry

    lax.fori_loop(0, (n_groups + 2) // 3, step, 0)
    drain_scatters((n_groups - 1) % 3)
    plsc.subcore_barrier()

    @pl.when(s == 0)
    def _writeout():
        pltpu.sync_copy(agg_sh, out_hbm.at[c])


def _sc_scatter_add(ne, dst_f, n_nodes, n_edges):
    per_w = n_edges // NW
    return pl.kernel(
        _scatter_body,
        out_type=jax.ShapeDtypeStruct((NC, n_nodes, EDGE_DIM), jnp.float32),
        mesh=_sc_mesh(),
        compiler_params=_SC_PARAMS,
        scratch_types=[
            pltpu.VMEM_SHARED((n_nodes, EDGE_DIM), jnp.float32),
            pltpu.VMEM((per_w,), jnp.int32),
            pltpu.VMEM((SUBS * CH, EDGE_DIM), jnp.float32),
            pltpu.VMEM((SUBS * CH, EDGE_DIM), jnp.float32),
            pltpu.VMEM((SUBS * CH, EDGE_DIM), jnp.float32),
            pltpu.VMEM((80, EDGE_DIM), jnp.float32),
            pltpu.SemaphoreType.DMA,
            pltpu.SemaphoreType.DMA,
            pltpu.SemaphoreType.DMA,
            pltpu.SemaphoreType.DMA,
            pltpu.SemaphoreType.DMA,
            pltpu.SemaphoreType.DMA,
        ],
    )(ne, dst_f)


def _node_body(x_ref, ap_ref, w1x_ref, w1a_ref, b1_ref, w2_ref,
               wrx_ref, wra_ref, bz_ref, gam_ref, bet_ref, out_ref):
    xb = x_ref[...]
    agg = ap_ref[0] + ap_ref[1]
    h = (jnp.dot(xb, w1x_ref[...], preferred_element_type=jnp.float32)
         + jnp.dot(agg, w1a_ref[...], preferred_element_type=jnp.float32)
         + b1_ref[...])
    y = h * (1.0 / (1.0 + jnp.exp(-h)))
    z = (jnp.dot(y, w2_ref[...], preferred_element_type=jnp.float32)
         + jnp.dot(xb, wrx_ref[...], preferred_element_type=jnp.float32)
         + jnp.dot(agg, wra_ref[...], preferred_element_type=jnp.float32)
         + bz_ref[...])
    mu = jnp.mean(z, axis=1, keepdims=True)
    zc = z - mu
    var = jnp.mean(zc * zc, axis=1, keepdims=True)
    out_ref[...] = zc * lax.rsqrt(var + _EPS) * gam_ref[...] + bet_ref[...]


def _node_mlp_fused(x, aggp, w1x, w1a, b1, w2, wrx, wra, bz, gam, bet):
    n = x.shape[0]
    blk = 1000
    hd = w2.shape[0]
    return pl.pallas_call(
        _node_body,
        grid=(n // blk,),
        in_specs=[
            pl.BlockSpec((blk, NODE_DIM), lambda i: (i, 0)),
            pl.BlockSpec((NC, blk, EDGE_DIM), lambda i: (0, i, 0)),
            pl.BlockSpec((NODE_DIM, hd), lambda i: (0, 0)),
            pl.BlockSpec((EDGE_DIM, hd), lambda i: (0, 0)),
            pl.BlockSpec((1, hd), lambda i: (0, 0)),
            pl.BlockSpec((hd, NODE_DIM), lambda i: (0, 0)),
            pl.BlockSpec((NODE_DIM, NODE_DIM), lambda i: (0, 0)),
            pl.BlockSpec((EDGE_DIM, NODE_DIM), lambda i: (0, 0)),
            pl.BlockSpec((1, NODE_DIM), lambda i: (0, 0)),
            pl.BlockSpec((1, NODE_DIM), lambda i: (0, 0)),
            pl.BlockSpec((1, NODE_DIM), lambda i: (0, 0)),
        ],
        out_specs=pl.BlockSpec((blk, NODE_DIM), lambda i: (i, 0)),
        out_shape=jax.ShapeDtypeStruct((n, NODE_DIM), jnp.float32),
    )(x, aggp, w1x, w1a, b1, w2, wrx, wra, bz, gam, bet)


def kernel(x, edge_attr, edge_index, edge_mlp, node_mlp):
    n_nodes = x.shape[0]
    n_edges = edge_attr.shape[0]

    src_f = edge_index[0].astype(jnp.int32)
    dst_f = edge_index[1].astype(jnp.int32)

    w1, wr = edge_mlp["W1"], edge_mlp["Wr"]
    ps, pd, rs, rd = _node_proj(
        x,
        w1[EDGE_DIM:EDGE_DIM + NODE_DIM],
        w1[EDGE_DIM + NODE_DIM:],
        wr[EDGE_DIM:EDGE_DIM + NODE_DIM],
        wr[EDGE_DIM + NODE_DIM:],
    )
    r = _sc_gather_add(rs, rd, src_f, dst_f, n_edges, EDGE_DIM)
    h = _sc_gather_add(ps, pd, src_f, dst_f, n_edges, NODE_DIM)

    new_edge = _edge_mlp_fused(
        h, r, edge_attr,
        w1[:EDGE_DIM],
        edge_mlp["b1"].reshape(1, -1),
        edge_mlp["W2"],
        wr[:EDGE_DIM],
        (edge_mlp["b2"] + edge_mlp["br"]).reshape(1, -1),
        edge_mlp["gamma"].reshape(1, -1),
        edge_mlp["beta"].reshape(1, -1),
    )

    aggp = _sc_scatter_add(new_edge, dst_f, n_nodes, n_edges)

    nw1, nwr = node_mlp["W1"], node_mlp["Wr"]
    new_x = _node_mlp_fused(
        x, aggp,
        nw1[:NODE_DIM], nw1[NODE_DIM:],
        node_mlp["b1"].reshape(1, -1),
        node_mlp["W2"],
        nwr[:NODE_DIM], nwr[NODE_DIM:],
        (node_mlp["b2"] + node_mlp["br"]).reshape(1, -1),
        node_mlp["gamma"].reshape(1, -1),
        node_mlp["beta"].reshape(1, -1),
    )
    return new_x, new_edge


# speedup vs baseline: 1.3400x; 1.3400x over previous
import jax
import jax.numpy as jnp
from jax import lax
from jax.experimental import pallas as pl
from jax.experimental.pallas import tpu as pltpu
from jax.experimental.pallas import tpu_sc as plsc

NODE_DIM = 128
EDGE_DIM = 16

NC, NS = 2, 16
NW = NC * NS
CH = 80
SUBS = 5

_EPS = 1e-5

_SC_PARAMS = pltpu.CompilerParams(use_tc_tiling_on_sc=False)


def _sc_mesh():
    return plsc.VectorSubcoreMesh(core_axis_name="c", subcore_axis_name="s",
                                  num_cores=NC, num_subcores=NS)


def _proj_body(x_ref, w1s_ref, w1d_ref, wrs_ref, wrd_ref,
               ps_ref, pd_ref, rs_ref, rd_ref):
    xb = x_ref[...]
    ps_ref[...] = jnp.dot(xb, w1s_ref[...], preferred_element_type=jnp.float32)
    pd_ref[...] = jnp.dot(xb, w1d_ref[...], preferred_element_type=jnp.float32)
    rs_ref[...] = jnp.dot(xb, wrs_ref[...], preferred_element_type=jnp.float32)
    rd_ref[...] = jnp.dot(xb, wrd_ref[...], preferred_element_type=jnp.float32)


def _node_proj(x, w1s, w1d, wrs, wrd):
    n = x.shape[0]
    blk = 1000
    return pl.pallas_call(
        _proj_body,
        grid=(n // blk,),
        in_specs=[
            pl.BlockSpec((blk, NODE_DIM), lambda i: (i, 0)),
            pl.BlockSpec((NODE_DIM, NODE_DIM), lambda i: (0, 0)),
            pl.BlockSpec((NODE_DIM, NODE_DIM), lambda i: (0, 0)),
            pl.BlockSpec((NODE_DIM, EDGE_DIM), lambda i: (0, 0)),
            pl.BlockSpec((NODE_DIM, EDGE_DIM), lambda i: (0, 0)),
        ],
        out_specs=[
            pl.BlockSpec((blk, NODE_DIM), lambda i: (i, 0)),
            pl.BlockSpec((blk, NODE_DIM), lambda i: (i, 0)),
            pl.BlockSpec((blk, EDGE_DIM), lambda i: (i, 0)),
            pl.BlockSpec((blk, EDGE_DIM), lambda i: (i, 0)),
        ],
        out_shape=[
            jax.ShapeDtypeStruct((n, NODE_DIM), jnp.float32),
            jax.ShapeDtypeStruct((n, NODE_DIM), jnp.float32),
            jax.ShapeDtypeStruct((n, EDGE_DIM), jnp.float32),
            jax.ShapeDtypeStruct((n, EDGE_DIM), jnp.float32),
        ],
    )(x, w1s, w1d, wrs, wrd)


def _make_gather_body(width):
    def body(ps_hbm, pd_hbm, src_hbm, dst_hbm, h_hbm,
             idx_s, idx_d, bs0, bd0, bs1, bd1, bw0, bw1,
             sg0, sg1, sw0, sw1):
        per_w = idx_s.shape[0]
        n_chunks = per_w // CH
        c = lax.axis_index("c")
        s = lax.axis_index("s")
        wid = c * NS + s
        base = wid * per_w
        pltpu.sync_copy(src_hbm.at[pl.ds(base, per_w)], idx_s)
        pltpu.sync_copy(dst_hbm.at[pl.ds(base, per_w)], idx_d)
        bs = (bs0, bs1)
        bd = (bd0, bd1)
        bw = (bw0, bw1)
        sg = (sg0, sg1)
        sw = (sw0, sw1)

        def fire(t, b):
            tsl = pl.ds(t * CH, CH)
            pltpu.async_copy(ps_hbm.at[idx_s.at[tsl]], bs[b], sg[b])
            pltpu.async_copy(pd_hbm.at[idx_d.at[tsl]], bd[b], sg[b])

        def process(t, b):
            tsl = pl.ds(t * CH, CH)
            pltpu.make_async_copy(ps_hbm.at[idx_s.at[tsl]], bs[b],
                                  sg[b]).wait()
            pltpu.make_async_copy(pd_hbm.at[idx_d.at[tsl]], bd[b],
                                  sg[b]).wait()

            @pl.when(t >= 2)
            def _():
                pltpu.make_async_copy(bw[b], h_hbm.at[pl.ds(base, CH)],
                                      sw[b]).wait()

            def row(i, c2):
                for j in range(width // 16):
                    sl = pl.ds(j * 16, 16)
                    bw[b][i, sl] = bs[b][i, sl] + bd[b][i, sl]
                return c2

            lax.fori_loop(0, CH, row, 0)
            pltpu.async_copy(bw[b], h_hbm.at[pl.ds(base + t * CH, CH)], sw[b])

            @pl.when(t + 2 < n_chunks)
            def _():
                fire(t + 2, b)

        fire(0, 0)
        fire(1, 1)

        def step(tt, carry):
            for b in range(2):
                t = 2 * tt + b

                @pl.when(t < n_chunks)
                def _():
                    process(t, b)

            return carry

        lax.fori_loop(0, (n_chunks + 1) // 2, step, 0)
        pltpu.make_async_copy(bw0, h_hbm.at[pl.ds(base, CH)], sw0).wait()
        pltpu.make_async_copy(bw1, h_hbm.at[pl.ds(base, CH)], sw1).wait()

    return body


def _sc_gather_add(ps, pd, src_f, dst_f, n_edges, width):
    per_w = n_edges // NW
    params = None if width == NODE_DIM else _SC_PARAMS
    return pl.kernel(
        _make_gather_body(width),
        out_type=jax.ShapeDtypeStruct((n_edges, width), jnp.float32),
        mesh=_sc_mesh(),
        compiler_params=params,
        scratch_types=[
            pltpu.VMEM((per_w,), jnp.int32),
            pltpu.VMEM((per_w,), jnp.int32),
            pltpu.VMEM((CH, width), jnp.float32),
            pltpu.VMEM((CH, width), jnp.float32),
            pltpu.VMEM((CH, width), jnp.float32),
            pltpu.VMEM((CH, width), jnp.float32),
            pltpu.VMEM((CH, width), jnp.float32),
            pltpu.VMEM((CH, width), jnp.float32),
            pltpu.SemaphoreType.DMA,
            pltpu.SemaphoreType.DMA,
            pltpu.SemaphoreType.DMA,
            pltpu.SemaphoreType.DMA,
        ],
    )(ps, pd, src_f, dst_f)


def _edge_body(h_ref, r_ref, ea_ref, w1e_ref, b1_ref, w2_ref, wre_ref,
               bz_ref, gam_ref, bet_ref, out_ref):
    ea = ea_ref[...]
    h = (h_ref[...]
         + jnp.dot(ea, w1e_ref[...], preferred_element_type=jnp.float32)
         + b1_ref[...])
    y = h * (1.0 / (1.0 + jnp.exp(-h)))
    z = (jnp.dot(y, w2_ref[...], preferred_element_type=jnp.float32)
         + r_ref[...]
         + jnp.dot(ea, wre_ref[...], preferred_element_type=jnp.float32)
         + bz_ref[...])
    mu = jnp.mean(z, axis=1, keepdims=True)
    zc = z - mu
    var = jnp.mean(zc * zc, axis=1, keepdims=True)
    out_ref[...] = zc * lax.rsqrt(var + _EPS) * gam_ref[...] + bet_ref[...]


def _edge_mlp_fused(h, r, ea, w1e, b1, w2, wre, bz, gam, bet):
    n_edges = h.shape[0]
    blk = 2000
    return pl.pallas_call(
        _edge_body,
        grid=(n_edges // blk,),
        in_specs=[
            pl.BlockSpec((blk, NODE_DIM), lambda i: (i, 0)),
            pl.BlockSpec((blk, EDGE_DIM), lambda i: (i, 0)),
            pl.BlockSpec((blk, EDGE_DIM), lambda i: (i, 0)),
            pl.BlockSpec((EDGE_DIM, NODE_DIM), lambda i: (0, 0)),
            pl.BlockSpec((1, NODE_DIM), lambda i: (0, 0)),
            pl.BlockSpec((NODE_DIM, EDGE_DIM), lambda i: (0, 0)),
            pl.BlockSpec((EDGE_DIM, EDGE_DIM), lambda i: (0, 0)),
            pl.BlockSpec((1, EDGE_DIM), lambda i: (0, 0)),
            pl.BlockSpec((1, EDGE_DIM), lambda i: (0, 0)),
            pl.BlockSpec((1, EDGE_DIM), lambda i: (0, 0)),
        ],
        out_specs=pl.BlockSpec((blk, EDGE_DIM), lambda i: (i, 0)),
        out_shape=jax.ShapeDtypeStruct((n_edges, EDGE_DIM), jnp.float32),
    )(h, r, ea, w1e, b1, w2, wre, bz, gam, bet)


def _scatter_body(ne_hbm, dst_hbm, out_hbm, agg_sh, idx_v,
                  rows0, rows1, rows2, zb, sl0, sl1, sl2, ss0, ss1, ss2):
    per_w = idx_v.shape[0]
    n_chunks = per_w // CH
    n_groups = n_chunks // SUBS
    grp = SUBS * CH
    n_nodes = agg_sh.shape[0]
    zrows = zb.shape[0]
    n_zcopies = n_nodes // zrows
    c = lax.axis_index("c")
    s = lax.axis_index("s")
    wid = c * NS + s
    base = wid * per_w
    rows = (rows0, rows1, rows2)
    sl = (sl0, sl1, sl2)
    ss = (ss0, ss1, ss2)

    def zr(i, carry):
        zb[i, :] = jnp.zeros((16,), jnp.float32)
        return carry

    lax.fori_loop(0, zrows, zr, 0)

    def zcopy(j, carry):
        k = s + NS * j

        @pl.when(k < n_zcopies)
        def _():
            pltpu.sync_copy(zb, agg_sh.at[pl.ds(k * zrows, zrows)])

        return carry

    lax.fori_loop(0, (n_zcopies + NS - 1) // NS, zcopy, 0)
    pltpu.sync_copy(dst_hbm.at[pl.ds(base, per_w)], idx_v)
    plsc.subcore_barrier()

    def fire(g, b):
        pltpu.async_copy(ne_hbm.at[pl.ds(base + g * grp, grp)], rows[b], sl[b])

    def drain_scatters(b):
        for j in range(SUBS):
            pltpu.make_async_copy(rows[b].at[pl.ds(j * CH, CH)],
                                  agg_sh.at[idx_v.at[pl.ds(0, CH)]],
                                  ss[b]).wait()

    def process(g, b, bn):
        pltpu.make_async_copy(ne_hbm.at[pl.ds(base, grp)], rows[b],
                              sl[b]).wait()
        for j in range(SUBS):
            isl = pl.ds(g * grp + j * CH, CH)
            pltpu.async_copy(rows[b].at[pl.ds(j * CH, CH)],
                             agg_sh.at[idx_v.at[isl]], ss[b], add=True)

        @pl.when(g >= 1)
        def _():
            drain_scatters(bn)

        @pl.when(g + 2 < n_groups)
        def _():
            fire(g + 2, bn)

    fire(0, 0)
    fire(1, 1)

    def step(gg, carry):
        for b in range(3):
            g = 3 * gg + b

            @pl.when(g < n_groups)
            def _():
                process(g, b, (b + 2) % 3)

        return carry

    lax.fori_loop(0, (n_groups + 2) // 3, step, 0)
    drain_scatters((n_groups - 1) % 3)
    plsc.subcore_barrier()

    @pl.when(s == 0)
    def _writeout():
        pltpu.sync_copy(agg_sh, out_hbm.at[c])


def _sc_scatter_add(ne, dst_f, n_nodes, n_edges):
    per_w = n_edges // NW
    return pl.kernel(
        _scatter_body,
        out_type=jax.ShapeDtypeStruct((NC, n_nodes, EDGE_DIM), jnp.float32),
        mesh=_sc_mesh(),
        compiler_params=_SC_PARAMS,
        scratch_types=[
            pltpu.VMEM_SHARED((n_nodes, EDGE_DIM), jnp.float32),
            pltpu.VMEM((per_w,), jnp.int32),
            pltpu.VMEM((SUBS * CH, EDGE_DIM), jnp.float32),
            pltpu.VMEM((SUBS * CH, EDGE_DIM), jnp.float32),
            pltpu.VMEM((SUBS * CH, EDGE_DIM), jnp.float32),
            pltpu.VMEM((80, EDGE_DIM), jnp.float32),
            pltpu.SemaphoreType.DMA,
            pltpu.SemaphoreType.DMA,
            pltpu.SemaphoreType.DMA,
            pltpu.SemaphoreType.DMA,
            pltpu.SemaphoreType.DMA,
            pltpu.SemaphoreType.DMA,
        ],
    )(ne, dst_f)


def _node_body(x_ref, ap_ref, w1x_ref, w1a_ref, b1_ref, w2_ref,
               wrx_ref, wra_ref, bz_ref, gam_ref, bet_ref, out_ref):
    xb = x_ref[...]
    agg = ap_ref[0] + ap_ref[1]
    h = (jnp.dot(xb, w1x_ref[...], preferred_element_type=jnp.float32)
         + jnp.dot(agg, w1a_ref[...], preferred_element_type=jnp.float32)
         + b1_ref[...])
    y = h * (1.0 / (1.0 + jnp.exp(-h)))
    z = (jnp.dot(y, w2_ref[...], preferred_element_type=jnp.float32)
         + jnp.dot(xb, wrx_ref[...], preferred_element_type=jnp.float32)
         + jnp.dot(agg, wra_ref[...], preferred_element_type=jnp.float32)
         + bz_ref[...])
    mu = jnp.mean(z, axis=1, keepdims=True)
    zc = z - mu
    var = jnp.mean(zc * zc, axis=1, keepdims=True)
    out_ref[...] = zc * lax.rsqrt(var + _EPS) * gam_ref[...] + bet_ref[...]


def _node_mlp_fused(x, aggp, w1x, w1a, b1, w2, wrx, wra, bz, gam, bet):
    n = x.shape[0]
    blk = 1000
    hd = w2.shape[0]
    return pl.pallas_call(
        _node_body,
        grid=(n // blk,),
        in_specs=[
            pl.BlockSpec((blk, NODE_DIM), lambda i: (i, 0)),
            pl.BlockSpec((NC, blk, EDGE_DIM), lambda i: (0, i, 0)),
            pl.BlockSpec((NODE_DIM, hd), lambda i: (0, 0)),
            pl.BlockSpec((EDGE_DIM, hd), lambda i: (0, 0)),
            pl.BlockSpec((1, hd), lambda i: (0, 0)),
            pl.BlockSpec((hd, NODE_DIM), lambda i: (0, 0)),
            pl.BlockSpec((NODE_DIM, NODE_DIM), lambda i: (0, 0)),
            pl.BlockSpec((EDGE_DIM, NODE_DIM), lambda i: (0, 0)),
            pl.BlockSpec((1, NODE_DIM), lambda i: (0, 0)),
            pl.BlockSpec((1, NODE_DIM), lambda i: (0, 0)),
            pl.BlockSpec((1, NODE_DIM), lambda i: (0, 0)),
        ],
        out_specs=pl.BlockSpec((blk, NODE_DIM), lambda i: (i, 0)),
        out_shape=jax.ShapeDtypeStruct((n, NODE_DIM), jnp.float32),
    )(x, aggp, w1x, w1a, b1, w2, wrx, wra, bz, gam, bet)


def kernel(x, edge_attr, edge_index, edge_mlp, node_mlp):
    n_nodes = x.shape[0]
    n_edges = edge_attr.shape[0]

    src_f = edge_index[0].astype(jnp.int32)
    dst_f = edge_index[1].astype(jnp.int32)

    w1, wr = edge_mlp["W1"], edge_mlp["Wr"]
    ps, pd, rs, rd = _node_proj(
        x,
        w1[EDGE_DIM:EDGE_DIM + NODE_DIM],
        w1[EDGE_DIM + NODE_DIM:],
        wr[EDGE_DIM:EDGE_DIM + NODE_DIM],
        wr[EDGE_DIM + NODE_DIM:],
    )
    r = _sc_gather_add(rs, rd, src_f, dst_f, n_edges, EDGE_DIM)
    h = _sc_gather_add(ps, pd, src_f, dst_f, n_edges, NODE_DIM)

    new_edge = _edge_mlp_fused(
        h, r, edge_attr,
        w1[:EDGE_DIM],
        edge_mlp["b1"].reshape(1, -1),
        edge_mlp["W2"],
        wr[:EDGE_DIM],
        (edge_mlp["b2"] + edge_mlp["br"]).reshape(1, -1),
        edge_mlp["gamma"].reshape(1, -1),
        edge_mlp["beta"].reshape(1, -1),
    )

    aggp = _sc_scatter_add(new_edge, dst_f, n_nodes, n_edges)

    nw1, nwr = node_mlp["W1"], node_mlp["Wr"]
    new_x = _node_mlp_fused(
        x, aggp,
        nw1[:NODE_DIM], nw1[NODE_DIM:],
        node_mlp["b1"].reshape(1, -1),
        node_mlp["W2"],
        nwr[:NODE_DIM], nwr[NODE_DIM:],
        (node_mlp["b2"] + node_mlp["br"]).reshape(1, -1),
        node_mlp["gamma"].reshape(1, -1),
        node_mlp["beta"].reshape(1, -1),
    )
    return new_x, new_edge


# speedup vs baseline: 1.4175x; 1.0578x over previous
import jax
import jax.numpy as jnp
from jax import lax
from jax.experimental import pallas as pl
from jax.experimental.pallas import tpu as pltpu
from jax.experimental.pallas import tpu_sc as plsc

NODE_DIM = 128
EDGE_DIM = 16

NC, NS = 2, 16
NW = NC * NS
CH = 80
SUBS = 5

_EPS = 1e-5

_SC_PARAMS = pltpu.CompilerParams(use_tc_tiling_on_sc=False)


def _sc_mesh():
    return plsc.VectorSubcoreMesh(core_axis_name="c", subcore_axis_name="s",
                                  num_cores=NC, num_subcores=NS)


def _proj_body(x_ref, w1s_ref, w1d_ref, wrs_ref, wrd_ref,
               ps_ref, pd_ref, rs_ref, rd_ref):
    xb = x_ref[...]
    ps_ref[...] = jnp.dot(xb, w1s_ref[...], preferred_element_type=jnp.float32)
    pd_ref[...] = jnp.dot(xb, w1d_ref[...], preferred_element_type=jnp.float32)
    rs_ref[...] = jnp.dot(xb, wrs_ref[...], preferred_element_type=jnp.float32)
    rd_ref[...] = jnp.dot(xb, wrd_ref[...], preferred_element_type=jnp.float32)


def _node_proj(x, w1s, w1d, wrs, wrd):
    n = x.shape[0]
    blk = 1000
    return pl.pallas_call(
        _proj_body,
        grid=(n // blk,),
        in_specs=[
            pl.BlockSpec((blk, NODE_DIM), lambda i: (i, 0)),
            pl.BlockSpec((NODE_DIM, NODE_DIM), lambda i: (0, 0)),
            pl.BlockSpec((NODE_DIM, NODE_DIM), lambda i: (0, 0)),
            pl.BlockSpec((NODE_DIM, EDGE_DIM), lambda i: (0, 0)),
            pl.BlockSpec((NODE_DIM, EDGE_DIM), lambda i: (0, 0)),
        ],
        out_specs=[
            pl.BlockSpec((blk, NODE_DIM), lambda i: (i, 0)),
            pl.BlockSpec((blk, NODE_DIM), lambda i: (i, 0)),
            pl.BlockSpec((blk, EDGE_DIM), lambda i: (i, 0)),
            pl.BlockSpec((blk, EDGE_DIM), lambda i: (i, 0)),
        ],
        out_shape=[
            jax.ShapeDtypeStruct((n, NODE_DIM), jnp.float32),
            jax.ShapeDtypeStruct((n, NODE_DIM), jnp.float32),
            jax.ShapeDtypeStruct((n, EDGE_DIM), jnp.float32),
            jax.ShapeDtypeStruct((n, EDGE_DIM), jnp.float32),
        ],
    )(x, w1s, w1d, wrs, wrd)


def _make_gather_body(width):
    def body(ps_hbm, pd_hbm, src_hbm, dst_hbm, h_hbm,
             idx_s, idx_d, bs0, bd0, bs1, bd1, bw0, bw1,
             sg0, sg1, sw0, sw1):
        per_w = idx_s.shape[0]
        n_chunks = per_w // CH
        c = lax.axis_index("c")
        s = lax.axis_index("s")
        wid = c * NS + s
        base = wid * per_w
        pltpu.sync_copy(src_hbm.at[pl.ds(base, per_w)], idx_s)
        pltpu.sync_copy(dst_hbm.at[pl.ds(base, per_w)], idx_d)
        bs = (bs0, bs1)
        bd = (bd0, bd1)
        bw = (bw0, bw1)
        sg = (sg0, sg1)
        sw = (sw0, sw1)

        def fire(t, b):
            tsl = pl.ds(t * CH, CH)
            pltpu.async_copy(ps_hbm.at[idx_s.at[tsl]], bs[b], sg[b])
            pltpu.async_copy(pd_hbm.at[idx_d.at[tsl]], bd[b], sg[b])

        def process(t, b):
            tsl = pl.ds(t * CH, CH)
            pltpu.make_async_copy(ps_hbm.at[idx_s.at[tsl]], bs[b],
                                  sg[b]).wait()
            pltpu.make_async_copy(pd_hbm.at[idx_d.at[tsl]], bd[b],
                                  sg[b]).wait()

            @pl.when(t >= 2)
            def _():
                pltpu.make_async_copy(bw[b], h_hbm.at[pl.ds(base, CH)],
                                      sw[b]).wait()

            def row(i, c2):
                for j in range(width // 16):
                    sl = pl.ds(j * 16, 16)
                    bw[b][i, sl] = bs[b][i, sl] + bd[b][i, sl]
                return c2

            lax.fori_loop(0, CH, row, 0)
            pltpu.async_copy(bw[b], h_hbm.at[pl.ds(base + t * CH, CH)], sw[b])

            @pl.when(t + 2 < n_chunks)
            def _():
                fire(t + 2, b)

        fire(0, 0)
        fire(1, 1)

        def step(tt, carry):
            for b in range(2):
                t = 2 * tt + b

                @pl.when(t < n_chunks)
                def _():
                    process(t, b)

            return carry

        lax.fori_loop(0, (n_chunks + 1) // 2, step, 0)
        pltpu.make_async_copy(bw0, h_hbm.at[pl.ds(base, CH)], sw0).wait()
        pltpu.make_async_copy(bw1, h_hbm.at[pl.ds(base, CH)], sw1).wait()

    return body


def _sc_gather_add(ps, pd, src_f, dst_f, n_edges, width):
    per_w = n_edges // NW
    params = None if width == NODE_DIM else _SC_PARAMS
    return pl.kernel(
        _make_gather_body(width),
        out_type=jax.ShapeDtypeStruct((n_edges, width), jnp.float32),
        mesh=_sc_mesh(),
        compiler_params=params,
        scratch_types=[
            pltpu.VMEM((per_w,), jnp.int32),
            pltpu.VMEM((per_w,), jnp.int32),
            pltpu.VMEM((CH, width), jnp.float32),
            pltpu.VMEM((CH, width), jnp.float32),
            pltpu.VMEM((CH, width), jnp.float32),
            pltpu.VMEM((CH, width), jnp.float32),
            pltpu.VMEM((CH, width), jnp.float32),
            pltpu.VMEM((CH, width), jnp.float32),
            pltpu.SemaphoreType.DMA,
            pltpu.SemaphoreType.DMA,
            pltpu.SemaphoreType.DMA,
            pltpu.SemaphoreType.DMA,
        ],
    )(ps, pd, src_f, dst_f)


def _edge_body(h_ref, r_ref, ea_ref, w1e_ref, b1_ref, w2_ref, wre_ref,
               bz_ref, gam_ref, bet_ref, out_ref):
    ea = ea_ref[...]
    h = (h_ref[...]
         + jnp.dot(ea, w1e_ref[...], preferred_element_type=jnp.float32)
         + b1_ref[...])
    y = h * (1.0 / (1.0 + jnp.exp(-h)))
    z = (jnp.dot(y, w2_ref[...], preferred_element_type=jnp.float32)
         + r_ref[...]
         + jnp.dot(ea, wre_ref[...], preferred_element_type=jnp.float32)
         + bz_ref[...])
    mu = jnp.mean(z, axis=1, keepdims=True)
    zc = z - mu
    var = jnp.mean(zc * zc, axis=1, keepdims=True)
    out_ref[...] = zc * lax.rsqrt(var + _EPS) * gam_ref[...] + bet_ref[...]


def _edge_mlp_fused(h, r, ea, w1e, b1, w2, wre, bz, gam, bet):
    n_edges = h.shape[0]
    blk = 4000
    return pl.pallas_call(
        _edge_body,
        grid=(n_edges // blk,),
        in_specs=[
            pl.BlockSpec((blk, NODE_DIM), lambda i: (i, 0)),
            pl.BlockSpec((blk, EDGE_DIM), lambda i: (i, 0)),
            pl.BlockSpec((blk, EDGE_DIM), lambda i: (i, 0)),
            pl.BlockSpec((EDGE_DIM, NODE_DIM), lambda i: (0, 0)),
            pl.BlockSpec((1, NODE_DIM), lambda i: (0, 0)),
            pl.BlockSpec((NODE_DIM, EDGE_DIM), lambda i: (0, 0)),
            pl.BlockSpec((EDGE_DIM, EDGE_DIM), lambda i: (0, 0)),
            pl.BlockSpec((1, EDGE_DIM), lambda i: (0, 0)),
            pl.BlockSpec((1, EDGE_DIM), lambda i: (0, 0)),
            pl.BlockSpec((1, EDGE_DIM), lambda i: (0, 0)),
        ],
        out_specs=pl.BlockSpec((blk, EDGE_DIM), lambda i: (i, 0)),
        out_shape=jax.ShapeDtypeStruct((n_edges, EDGE_DIM), jnp.float32),
    )(h, r, ea, w1e, b1, w2, wre, bz, gam, bet)


def _scatter_body(ne_hbm, dst_hbm, out_hbm, agg_sh, idx_v,
                  rows0, rows1, rows2, zb, sl0, sl1, sl2, ss0, ss1, ss2):
    per_w = idx_v.shape[0]
    n_chunks = per_w // CH
    n_groups = n_chunks // SUBS
    grp = SUBS * CH
    n_nodes = agg_sh.shape[0]
    zrows = zb.shape[0]
    n_zcopies = n_nodes // zrows
    c = lax.axis_index("c")
    s = lax.axis_index("s")
    wid = c * NS + s
    base = wid * per_w
    rows = (rows0, rows1, rows2)
    sl = (sl0, sl1, sl2)
    ss = (ss0, ss1, ss2)

    def zr(i, carry):
        zb[i, :] = jnp.zeros((16,), jnp.float32)
        return carry

    lax.fori_loop(0, zrows, zr, 0)

    def zcopy(j, carry):
        k = s + NS * j

        @pl.when(k < n_zcopies)
        def _():
            pltpu.sync_copy(zb, agg_sh.at[pl.ds(k * zrows, zrows)])

        return carry

    lax.fori_loop(0, (n_zcopies + NS - 1) // NS, zcopy, 0)
    pltpu.sync_copy(dst_hbm.at[pl.ds(base, per_w)], idx_v)
    plsc.subcore_barrier()

    def fire(g, b):
        pltpu.async_copy(ne_hbm.at[pl.ds(base + g * grp, grp)], rows[b], sl[b])

    def drain_scatters(b):
        for j in range(SUBS):
            pltpu.make_async_copy(rows[b].at[pl.ds(j * CH, CH)],
                                  agg_sh.at[idx_v.at[pl.ds(0, CH)]],
                                  ss[b]).wait()

    def process(g, b, bn):
        pltpu.make_async_copy(ne_hbm.at[pl.ds(base, grp)], rows[b],
                              sl[b]).wait()
        for j in range(SUBS):
            isl = pl.ds(g * grp + j * CH, CH)
            pltpu.async_copy(rows[b].at[pl.ds(j * CH, CH)],
                             agg_sh.at[idx_v.at[isl]], ss[b], add=True)

        @pl.when(g >= 1)
        def _():
            drain_scatters(bn)

        @pl.when(g + 2 < n_groups)
        def _():
            fire(g + 2, bn)

    fire(0, 0)
    fire(1, 1)

    def step(gg, carry):
        for b in range(3):
            g = 3 * gg + b

            @pl.when(g < n_groups)
            def _():
                process(g, b, (b + 2) % 3)

        return carry

    lax.fori_loop(0, (n_groups + 2) // 3, step, 0)
    drain_scatters((n_groups - 1) % 3)
    plsc.subcore_barrier()

    @pl.when(s == 0)
    def _writeout():
        pltpu.sync_copy(agg_sh, out_hbm.at[c])


def _sc_scatter_add(ne, dst_f, n_nodes, n_edges):
    per_w = n_edges // NW
    return pl.kernel(
        _scatter_body,
        out_type=jax.ShapeDtypeStruct((NC, n_nodes, EDGE_DIM), jnp.float32),
        mesh=_sc_mesh(),
        compiler_params=_SC_PARAMS,
        scratch_types=[
            pltpu.VMEM_SHARED((n_nodes, EDGE_DIM), jnp.float32),
            pltpu.VMEM((per_w,), jnp.int32),
            pltpu.VMEM((SUBS * CH, EDGE_DIM), jnp.float32),
            pltpu.VMEM((SUBS * CH, EDGE_DIM), jnp.float32),
            pltpu.VMEM((SUBS * CH, EDGE_DIM), jnp.float32),
            pltpu.VMEM((80, EDGE_DIM), jnp.float32),
            pltpu.SemaphoreType.DMA,
            pltpu.SemaphoreType.DMA,
            pltpu.SemaphoreType.DMA,
            pltpu.SemaphoreType.DMA,
            pltpu.SemaphoreType.DMA,
            pltpu.SemaphoreType.DMA,
        ],
    )(ne, dst_f)


def _node_body(x_ref, ap_ref, w1x_ref, w1a_ref, b1_ref, w2_ref,
               wrx_ref, wra_ref, bz_ref, gam_ref, bet_ref, out_ref):
    xb = x_ref[...]
    agg = ap_ref[0] + ap_ref[1]
    h = (jnp.dot(xb, w1x_ref[...], preferred_element_type=jnp.float32)
         + jnp.dot(agg, w1a_ref[...], preferred_element_type=jnp.float32)
         + b1_ref[...])
    y = h * (1.0 / (1.0 + jnp.exp(-h)))
    z = (jnp.dot(y, w2_ref[...], preferred_element_type=jnp.float32)
         + jnp.dot(xb, wrx_ref[...], preferred_element_type=jnp.float32)
         + jnp.dot(agg, wra_ref[...], preferred_element_type=jnp.float32)
         + bz_ref[...])
    mu = jnp.mean(z, axis=1, keepdims=True)
    zc = z - mu
    var = jnp.mean(zc * zc, axis=1, keepdims=True)
    out_ref[...] = zc * lax.rsqrt(var + _EPS) * gam_ref[...] + bet_ref[...]


def _node_mlp_fused(x, aggp, w1x, w1a, b1, w2, wrx, wra, bz, gam, bet):
    n = x.shape[0]
    blk = 1000
    hd = w2.shape[0]
    return pl.pallas_call(
        _node_body,
        grid=(n // blk,),
        in_specs=[
            pl.BlockSpec((blk, NODE_DIM), lambda i: (i, 0)),
            pl.BlockSpec((NC, blk, EDGE_DIM), lambda i: (0, i, 0)),
            pl.BlockSpec((NODE_DIM, hd), lambda i: (0, 0)),
            pl.BlockSpec((EDGE_DIM, hd), lambda i: (0, 0)),
            pl.BlockSpec((1, hd), lambda i: (0, 0)),
            pl.BlockSpec((hd, NODE_DIM), lambda i: (0, 0)),
            pl.BlockSpec((NODE_DIM, NODE_DIM), lambda i: (0, 0)),
            pl.BlockSpec((EDGE_DIM, NODE_DIM), lambda i: (0, 0)),
            pl.BlockSpec((1, NODE_DIM), lambda i: (0, 0)),
            pl.BlockSpec((1, NODE_DIM), lambda i: (0, 0)),
            pl.BlockSpec((1, NODE_DIM), lambda i: (0, 0)),
        ],
        out_specs=pl.BlockSpec((blk, NODE_DIM), lambda i: (i, 0)),
        out_shape=jax.ShapeDtypeStruct((n, NODE_DIM), jnp.float32),
    )(x, aggp, w1x, w1a, b1, w2, wrx, wra, bz, gam, bet)


def kernel(x, edge_attr, edge_index, edge_mlp, node_mlp):
    n_nodes = x.shape[0]
    n_edges = edge_attr.shape[0]

    src_f = edge_index[0].astype(jnp.int32)
    dst_f = edge_index[1].astype(jnp.int32)

    w1, wr = edge_mlp["W1"], edge_mlp["Wr"]
    ps, pd, rs, rd = _node_proj(
        x,
        w1[EDGE_DIM:EDGE_DIM + NODE_DIM],
        w1[EDGE_DIM + NODE_DIM:],
        wr[EDGE_DIM:EDGE_DIM + NODE_DIM],
        wr[EDGE_DIM + NODE_DIM:],
    )
    r = _sc_gather_add(rs, rd, src_f, dst_f, n_edges, EDGE_DIM)
    h = _sc_gather_add(ps, pd, src_f, dst_f, n_edges, NODE_DIM)

    new_edge = _edge_mlp_fused(
        h, r, edge_attr,
        w1[:EDGE_DIM],
        edge_mlp["b1"].reshape(1, -1),
        edge_mlp["W2"],
        wr[:EDGE_DIM],
        (edge_mlp["b2"] + edge_mlp["br"]).reshape(1, -1),
        edge_mlp["gamma"].reshape(1, -1),
        edge_mlp["beta"].reshape(1, -1),
    )

    aggp = _sc_scatter_add(new_edge, dst_f, n_nodes, n_edges)

    nw1, nwr = node_mlp["W1"], node_mlp["Wr"]
    new_x = _node_mlp_fused(
        x, aggp,
        nw1[:NODE_DIM], nw1[NODE_DIM:],
        node_mlp["b1"].reshape(1, -1),
        node_mlp["W2"],
        nwr[:NODE_DIM], nwr[NODE_DIM:],
        (node_mlp["b2"] + node_mlp["br"]).reshape(1, -1),
        node_mlp["gamma"].reshape(1, -1),
        node_mlp["beta"].reshape(1, -1),
    )
    return new_x, new_edge


# speedup vs baseline: 1.4525x; 1.0247x over previous
import jax
import jax.numpy as jnp
from jax import lax
from jax.experimental import pallas as pl
from jax.experimental.pallas import tpu as pltpu
from jax.experimental.pallas import tpu_sc as plsc

NODE_DIM = 128
EDGE_DIM = 16

NC, NS = 2, 16
NW = NC * NS
CH = 80
SUBS = 5

_EPS = 1e-5

_SC_PARAMS = pltpu.CompilerParams(use_tc_tiling_on_sc=False)


def _sc_mesh():
    return plsc.VectorSubcoreMesh(core_axis_name="c", subcore_axis_name="s",
                                  num_cores=NC, num_subcores=NS)


def _proj_body(x_ref, w1s_ref, w1d_ref, wrs_ref, wrd_ref,
               ps_ref, pd_ref, rs_ref, rd_ref):
    xb = x_ref[...]
    ps_ref[...] = jnp.dot(xb, w1s_ref[...], preferred_element_type=jnp.float32)
    pd_ref[...] = jnp.dot(xb, w1d_ref[...], preferred_element_type=jnp.float32)
    rs_ref[...] = jnp.dot(xb, wrs_ref[...], preferred_element_type=jnp.float32)
    rd_ref[...] = jnp.dot(xb, wrd_ref[...], preferred_element_type=jnp.float32)


def _node_proj(x, w1s, w1d, wrs, wrd):
    n = x.shape[0]
    blk = 1000
    return pl.pallas_call(
        _proj_body,
        grid=(n // blk,),
        in_specs=[
            pl.BlockSpec((blk, NODE_DIM), lambda i: (i, 0)),
            pl.BlockSpec((NODE_DIM, NODE_DIM), lambda i: (0, 0)),
            pl.BlockSpec((NODE_DIM, NODE_DIM), lambda i: (0, 0)),
            pl.BlockSpec((NODE_DIM, EDGE_DIM), lambda i: (0, 0)),
            pl.BlockSpec((NODE_DIM, EDGE_DIM), lambda i: (0, 0)),
        ],
        out_specs=[
            pl.BlockSpec((blk, NODE_DIM), lambda i: (i, 0)),
            pl.BlockSpec((blk, NODE_DIM), lambda i: (i, 0)),
            pl.BlockSpec((blk, EDGE_DIM), lambda i: (i, 0)),
            pl.BlockSpec((blk, EDGE_DIM), lambda i: (i, 0)),
        ],
        out_shape=[
            jax.ShapeDtypeStruct((n, NODE_DIM), jnp.float32),
            jax.ShapeDtypeStruct((n, NODE_DIM), jnp.float32),
            jax.ShapeDtypeStruct((n, EDGE_DIM), jnp.float32),
            jax.ShapeDtypeStruct((n, EDGE_DIM), jnp.float32),
        ],
    )(x, w1s, w1d, wrs, wrd)


def _make_gather_body(width):
    def body(ps_hbm, pd_hbm, src_hbm, dst_hbm, h_hbm,
             idx_s, idx_d, bs0, bd0, bs1, bd1, bw0, bw1,
             sg0, sg1, sw0, sw1):
        per_w = idx_s.shape[0]
        n_chunks = per_w // CH
        c = lax.axis_index("c")
        s = lax.axis_index("s")
        wid = c * NS + s
        base = wid * per_w
        pltpu.sync_copy(src_hbm.at[pl.ds(base, per_w)], idx_s)
        pltpu.sync_copy(dst_hbm.at[pl.ds(base, per_w)], idx_d)
        bs = (bs0, bs1)
        bd = (bd0, bd1)
        bw = (bw0, bw1)
        sg = (sg0, sg1)
        sw = (sw0, sw1)

        def fire(t, b):
            tsl = pl.ds(t * CH, CH)
            pltpu.async_copy(ps_hbm.at[idx_s.at[tsl]], bs[b], sg[b])
            pltpu.async_copy(pd_hbm.at[idx_d.at[tsl]], bd[b], sg[b])

        def process(t, b):
            tsl = pl.ds(t * CH, CH)
            pltpu.make_async_copy(ps_hbm.at[idx_s.at[tsl]], bs[b],
                                  sg[b]).wait()
            pltpu.make_async_copy(pd_hbm.at[idx_d.at[tsl]], bd[b],
                                  sg[b]).wait()

            @pl.when(t >= 2)
            def _():
                pltpu.make_async_copy(bw[b], h_hbm.at[pl.ds(base, CH)],
                                      sw[b]).wait()

            def row(i, c2):
                for j in range(width // 16):
                    sl = pl.ds(j * 16, 16)
                    bw[b][i, sl] = bs[b][i, sl] + bd[b][i, sl]
                return c2

            lax.fori_loop(0, CH, row, 0)
            pltpu.async_copy(bw[b], h_hbm.at[pl.ds(base + t * CH, CH)], sw[b])

            @pl.when(t + 2 < n_chunks)
            def _():
                fire(t + 2, b)

        fire(0, 0)
        fire(1, 1)

        def step(tt, carry):
            for b in range(2):
                t = 2 * tt + b

                @pl.when(t < n_chunks)
                def _():
                    process(t, b)

            return carry

        lax.fori_loop(0, (n_chunks + 1) // 2, step, 0)
        pltpu.make_async_copy(bw0, h_hbm.at[pl.ds(base, CH)], sw0).wait()
        pltpu.make_async_copy(bw1, h_hbm.at[pl.ds(base, CH)], sw1).wait()

    return body


def _sc_gather_add(ps, pd, src_f, dst_f, n_edges, width):
    per_w = n_edges // NW
    params = None if width == NODE_DIM else _SC_PARAMS
    return pl.kernel(
        _make_gather_body(width),
        out_type=jax.ShapeDtypeStruct((n_edges, width), jnp.float32),
        mesh=_sc_mesh(),
        compiler_params=params,
        scratch_types=[
            pltpu.VMEM((per_w,), jnp.int32),
            pltpu.VMEM((per_w,), jnp.int32),
            pltpu.VMEM((CH, width), jnp.float32),
            pltpu.VMEM((CH, width), jnp.float32),
            pltpu.VMEM((CH, width), jnp.float32),
            pltpu.VMEM((CH, width), jnp.float32),
            pltpu.VMEM((CH, width), jnp.float32),
            pltpu.VMEM((CH, width), jnp.float32),
            pltpu.SemaphoreType.DMA,
            pltpu.SemaphoreType.DMA,
            pltpu.SemaphoreType.DMA,
            pltpu.SemaphoreType.DMA,
        ],
    )(ps, pd, src_f, dst_f)


def _edge_body(h_ref, r_ref, ea_ref, w1e_ref, b1_ref, w2_ref, wre_ref,
               bz_ref, gam_ref, bet_ref, out_ref):
    ea = ea_ref[...]
    h = (h_ref[...]
         + jnp.dot(ea, w1e_ref[...], preferred_element_type=jnp.float32)
         + b1_ref[...])
    y = h * (1.0 / (1.0 + jnp.exp(-h)))
    z = (jnp.dot(y, w2_ref[...], preferred_element_type=jnp.float32)
         + r_ref[...]
         + jnp.dot(ea, wre_ref[...], preferred_element_type=jnp.float32)
         + bz_ref[...])
    mu = jnp.mean(z, axis=1, keepdims=True)
    zc = z - mu
    var = jnp.mean(zc * zc, axis=1, keepdims=True)
    out_ref[...] = zc * lax.rsqrt(var + _EPS) * gam_ref[...] + bet_ref[...]


def _edge_mlp_fused(h, r, ea, w1e, b1, w2, wre, bz, gam, bet):
    n_edges = h.shape[0]
    blk = 8000
    return pl.pallas_call(
        _edge_body,
        grid=(n_edges // blk,),
        in_specs=[
            pl.BlockSpec((blk, NODE_DIM), lambda i: (i, 0)),
            pl.BlockSpec((blk, EDGE_DIM), lambda i: (i, 0)),
            pl.BlockSpec((blk, EDGE_DIM), lambda i: (i, 0)),
            pl.BlockSpec((EDGE_DIM, NODE_DIM), lambda i: (0, 0)),
            pl.BlockSpec((1, NODE_DIM), lambda i: (0, 0)),
            pl.BlockSpec((NODE_DIM, EDGE_DIM), lambda i: (0, 0)),
            pl.BlockSpec((EDGE_DIM, EDGE_DIM), lambda i: (0, 0)),
            pl.BlockSpec((1, EDGE_DIM), lambda i: (0, 0)),
            pl.BlockSpec((1, EDGE_DIM), lambda i: (0, 0)),
            pl.BlockSpec((1, EDGE_DIM), lambda i: (0, 0)),
        ],
        out_specs=pl.BlockSpec((blk, EDGE_DIM), lambda i: (i, 0)),
        out_shape=jax.ShapeDtypeStruct((n_edges, EDGE_DIM), jnp.float32),
    )(h, r, ea, w1e, b1, w2, wre, bz, gam, bet)


def _scatter_body(ne_hbm, dst_hbm, out_hbm, agg_sh, idx_v,
                  rows0, rows1, rows2, zb, sl0, sl1, sl2, ss0, ss1, ss2):
    per_w = idx_v.shape[0]
    n_chunks = per_w // CH
    n_groups = n_chunks // SUBS
    grp = SUBS * CH
    n_nodes = agg_sh.shape[0]
    zrows = zb.shape[0]
    n_zcopies = n_nodes // zrows
    c = lax.axis_index("c")
    s = lax.axis_index("s")
    wid = c * NS + s
    base = wid * per_w
    rows = (rows0, rows1, rows2)
    sl = (sl0, sl1, sl2)
    ss = (ss0, ss1, ss2)

    def zr(i, carry):
        zb[i, :] = jnp.zeros((16,), jnp.float32)
        return carry

    lax.fori_loop(0, zrows, zr, 0)

    def zcopy(j, carry):
        k = s + NS * j

        @pl.when(k < n_zcopies)
        def _():
            pltpu.sync_copy(zb, agg_sh.at[pl.ds(k * zrows, zrows)])

        return carry

    lax.fori_loop(0, (n_zcopies + NS - 1) // NS, zcopy, 0)
    pltpu.sync_copy(dst_hbm.at[pl.ds(base, per_w)], idx_v)
    plsc.subcore_barrier()

    def fire(g, b):
        pltpu.async_copy(ne_hbm.at[pl.ds(base + g * grp, grp)], rows[b], sl[b])

    def drain_scatters(b):
        for j in range(SUBS):
            pltpu.make_async_copy(rows[b].at[pl.ds(j * CH, CH)],
                                  agg_sh.at[idx_v.at[pl.ds(0, CH)]],
                                  ss[b]).wait()

    def process(g, b, bn):
        pltpu.make_async_copy(ne_hbm.at[pl.ds(base, grp)], rows[b],
                              sl[b]).wait()
        for j in range(SUBS):
            isl = pl.ds(g * grp + j * CH, CH)
            pltpu.async_copy(rows[b].at[pl.ds(j * CH, CH)],
                             agg_sh.at[idx_v.at[isl]], ss[b], add=True)

        @pl.when(g >= 1)
        def _():
            drain_scatters(bn)

        @pl.when(g + 2 < n_groups)
        def _():
            fire(g + 2, bn)

    fire(0, 0)
    fire(1, 1)

    def step(gg, carry):
        for b in range(3):
            g = 3 * gg + b

            @pl.when(g < n_groups)
            def _():
                process(g, b, (b + 2) % 3)

        return carry

    lax.fori_loop(0, (n_groups + 2) // 3, step, 0)
    drain_scatters((n_groups - 1) % 3)
    plsc.subcore_barrier()

    @pl.when(s == 0)
    def _writeout():
        pltpu.sync_copy(agg_sh, out_hbm.at[c])


def _sc_scatter_add(ne, dst_f, n_nodes, n_edges):
    per_w = n_edges // NW
    return pl.kernel(
        _scatter_body,
        out_type=jax.ShapeDtypeStruct((NC, n_nodes, EDGE_DIM), jnp.float32),
        mesh=_sc_mesh(),
        compiler_params=_SC_PARAMS,
        scratch_types=[
            pltpu.VMEM_SHARED((n_nodes, EDGE_DIM), jnp.float32),
            pltpu.VMEM((per_w,), jnp.int32),
            pltpu.VMEM((SUBS * CH, EDGE_DIM), jnp.float32),
            pltpu.VMEM((SUBS * CH, EDGE_DIM), jnp.float32),
            pltpu.VMEM((SUBS * CH, EDGE_DIM), jnp.float32),
            pltpu.VMEM((80, EDGE_DIM), jnp.float32),
            pltpu.SemaphoreType.DMA,
            pltpu.SemaphoreType.DMA,
            pltpu.SemaphoreType.DMA,
            pltpu.SemaphoreType.DMA,
            pltpu.SemaphoreType.DMA,
            pltpu.SemaphoreType.DMA,
        ],
    )(ne, dst_f)


def _node_body(x_ref, ap_ref, w1x_ref, w1a_ref, b1_ref, w2_ref,
               wrx_ref, wra_ref, bz_ref, gam_ref, bet_ref, out_ref):
    xb = x_ref[...]
    agg = ap_ref[0] + ap_ref[1]
    h = (jnp.dot(xb, w1x_ref[...], preferred_element_type=jnp.float32)
         + jnp.dot(agg, w1a_ref[...], preferred_element_type=jnp.float32)
         + b1_ref[...])
    y = h * (1.0 / (1.0 + jnp.exp(-h)))
    z = (jnp.dot(y, w2_ref[...], preferred_element_type=jnp.float32)
         + jnp.dot(xb, wrx_ref[...], preferred_element_type=jnp.float32)
         + jnp.dot(agg, wra_ref[...], preferred_element_type=jnp.float32)
         + bz_ref[...])
    mu = jnp.mean(z, axis=1, keepdims=True)
    zc = z - mu
    var = jnp.mean(zc * zc, axis=1, keepdims=True)
    out_ref[...] = zc * lax.rsqrt(var + _EPS) * gam_ref[...] + bet_ref[...]


def _node_mlp_fused(x, aggp, w1x, w1a, b1, w2, wrx, wra, bz, gam, bet):
    n = x.shape[0]
    blk = 1000
    hd = w2.shape[0]
    return pl.pallas_call(
        _node_body,
        grid=(n // blk,),
        in_specs=[
            pl.BlockSpec((blk, NODE_DIM), lambda i: (i, 0)),
            pl.BlockSpec((NC, blk, EDGE_DIM), lambda i: (0, i, 0)),
            pl.BlockSpec((NODE_DIM, hd), lambda i: (0, 0)),
            pl.BlockSpec((EDGE_DIM, hd), lambda i: (0, 0)),
            pl.BlockSpec((1, hd), lambda i: (0, 0)),
            pl.BlockSpec((hd, NODE_DIM), lambda i: (0, 0)),
            pl.BlockSpec((NODE_DIM, NODE_DIM), lambda i: (0, 0)),
            pl.BlockSpec((EDGE_DIM, NODE_DIM), lambda i: (0, 0)),
            pl.BlockSpec((1, NODE_DIM), lambda i: (0, 0)),
            pl.BlockSpec((1, NODE_DIM), lambda i: (0, 0)),
            pl.BlockSpec((1, NODE_DIM), lambda i: (0, 0)),
        ],
        out_specs=pl.BlockSpec((blk, NODE_DIM), lambda i: (i, 0)),
        out_shape=jax.ShapeDtypeStruct((n, NODE_DIM), jnp.float32),
    )(x, aggp, w1x, w1a, b1, w2, wrx, wra, bz, gam, bet)


def kernel(x, edge_attr, edge_index, edge_mlp, node_mlp):
    n_nodes = x.shape[0]
    n_edges = edge_attr.shape[0]

    src_f = edge_index[0].astype(jnp.int32)
    dst_f = edge_index[1].astype(jnp.int32)

    w1, wr = edge_mlp["W1"], edge_mlp["Wr"]
    ps, pd, rs, rd = _node_proj(
        x,
        w1[EDGE_DIM:EDGE_DIM + NODE_DIM],
        w1[EDGE_DIM + NODE_DIM:],
        wr[EDGE_DIM:EDGE_DIM + NODE_DIM],
        wr[EDGE_DIM + NODE_DIM:],
    )
    r = _sc_gather_add(rs, rd, src_f, dst_f, n_edges, EDGE_DIM)
    h = _sc_gather_add(ps, pd, src_f, dst_f, n_edges, NODE_DIM)

    new_edge = _edge_mlp_fused(
        h, r, edge_attr,
        w1[:EDGE_DIM],
        edge_mlp["b1"].reshape(1, -1),
        edge_mlp["W2"],
        wr[:EDGE_DIM],
        (edge_mlp["b2"] + edge_mlp["br"]).reshape(1, -1),
        edge_mlp["gamma"].reshape(1, -1),
        edge_mlp["beta"].reshape(1, -1),
    )

    aggp = _sc_scatter_add(new_edge, dst_f, n_nodes, n_edges)

    nw1, nwr = node_mlp["W1"], node_mlp["Wr"]
    new_x = _node_mlp_fused(
        x, aggp,
        nw1[:NODE_DIM], nw1[NODE_DIM:],
        node_mlp["b1"].reshape(1, -1),
        node_mlp["W2"],
        nwr[:NODE_DIM], nwr[NODE_DIM:],
        (node_mlp["b2"] + node_mlp["br"]).reshape(1, -1),
        node_mlp["gamma"].reshape(1, -1),
        node_mlp["beta"].reshape(1, -1),
    )
    return new_x, new_edge


# speedup vs baseline: 1.5060x; 1.0369x over previous
import jax
import jax.numpy as jnp
from jax import lax
from jax.experimental import pallas as pl
from jax.experimental.pallas import tpu as pltpu
from jax.experimental.pallas import tpu_sc as plsc

NODE_DIM = 128
EDGE_DIM = 16

NC, NS = 2, 16
NW = NC * NS
CH = 80
SUBS = 5

_EPS = 1e-5

_SC_PARAMS = pltpu.CompilerParams(use_tc_tiling_on_sc=False)


def _sc_mesh():
    return plsc.VectorSubcoreMesh(core_axis_name="c", subcore_axis_name="s",
                                  num_cores=NC, num_subcores=NS)


def _proj_body(x_ref, w1s_ref, w1d_ref, wrs_ref, wrd_ref,
               ps_ref, pd_ref, rs_ref, rd_ref):
    xb = x_ref[...]
    ps_ref[...] = jnp.dot(xb, w1s_ref[...], preferred_element_type=jnp.float32)
    pd_ref[...] = jnp.dot(xb, w1d_ref[...], preferred_element_type=jnp.float32)
    rs_ref[...] = jnp.dot(xb, wrs_ref[...], preferred_element_type=jnp.float32)
    rd_ref[...] = jnp.dot(xb, wrd_ref[...], preferred_element_type=jnp.float32)


def _node_proj(x, w1s, w1d, wrs, wrd):
    n = x.shape[0]
    blk = 1000
    return pl.pallas_call(
        _proj_body,
        grid=(n // blk,),
        in_specs=[
            pl.BlockSpec((blk, NODE_DIM), lambda i: (i, 0)),
            pl.BlockSpec((NODE_DIM, NODE_DIM), lambda i: (0, 0)),
            pl.BlockSpec((NODE_DIM, NODE_DIM), lambda i: (0, 0)),
            pl.BlockSpec((NODE_DIM, EDGE_DIM), lambda i: (0, 0)),
            pl.BlockSpec((NODE_DIM, EDGE_DIM), lambda i: (0, 0)),
        ],
        out_specs=[
            pl.BlockSpec((blk, NODE_DIM), lambda i: (i, 0)),
            pl.BlockSpec((blk, NODE_DIM), lambda i: (i, 0)),
            pl.BlockSpec((blk, EDGE_DIM), lambda i: (i, 0)),
            pl.BlockSpec((blk, EDGE_DIM), lambda i: (i, 0)),
        ],
        out_shape=[
            jax.ShapeDtypeStruct((n, NODE_DIM), jnp.float32),
            jax.ShapeDtypeStruct((n, NODE_DIM), jnp.float32),
            jax.ShapeDtypeStruct((n, EDGE_DIM), jnp.float32),
            jax.ShapeDtypeStruct((n, EDGE_DIM), jnp.float32),
        ],
    )(x, w1s, w1d, wrs, wrd)


def _make_gather_body(width):
    def body(ps_hbm, pd_hbm, src_hbm, dst_hbm, h_hbm,
             idx_s, idx_d, bs0, bd0, bs1, bd1, bw0, bw1,
             sg0, sg1, sw0, sw1):
        per_w = idx_s.shape[0]
        n_chunks = per_w // CH
        c = lax.axis_index("c")
        s = lax.axis_index("s")
        wid = c * NS + s
        base = wid * per_w
        pltpu.sync_copy(src_hbm.at[pl.ds(base, per_w)], idx_s)
        pltpu.sync_copy(dst_hbm.at[pl.ds(base, per_w)], idx_d)
        bs = (bs0, bs1)
        bd = (bd0, bd1)
        bw = (bw0, bw1)
        sg = (sg0, sg1)
        sw = (sw0, sw1)

        def fire(t, b):
            tsl = pl.ds(t * CH, CH)
            pltpu.async_copy(ps_hbm.at[idx_s.at[tsl]], bs[b], sg[b])
            pltpu.async_copy(pd_hbm.at[idx_d.at[tsl]], bd[b], sg[b])

        def process(t, b):
            tsl = pl.ds(t * CH, CH)
            pltpu.make_async_copy(ps_hbm.at[idx_s.at[tsl]], bs[b],
                                  sg[b]).wait()
            pltpu.make_async_copy(pd_hbm.at[idx_d.at[tsl]], bd[b],
                                  sg[b]).wait()

            @pl.when(t >= 2)
            def _():
                pltpu.make_async_copy(bw[b], h_hbm.at[pl.ds(base, CH)],
                                      sw[b]).wait()

            def row(i, c2):
                for j in range(width // 16):
                    sl = pl.ds(j * 16, 16)
                    bw[b][i, sl] = bs[b][i, sl] + bd[b][i, sl]
                return c2

            lax.fori_loop(0, CH, row, 0)
            pltpu.async_copy(bw[b], h_hbm.at[pl.ds(base + t * CH, CH)], sw[b])

            @pl.when(t + 2 < n_chunks)
            def _():
                fire(t + 2, b)

        fire(0, 0)
        fire(1, 1)

        def step(tt, carry):
            for b in range(2):
                t = 2 * tt + b

                @pl.when(t < n_chunks)
                def _():
                    process(t, b)

            return carry

        lax.fori_loop(0, (n_chunks + 1) // 2, step, 0)
        pltpu.make_async_copy(bw0, h_hbm.at[pl.ds(base, CH)], sw0).wait()
        pltpu.make_async_copy(bw1, h_hbm.at[pl.ds(base, CH)], sw1).wait()

    return body


def _sc_gather_add(ps, pd, src_f, dst_f, n_edges, width):
    per_w = n_edges // NW
    params = None if width == NODE_DIM else _SC_PARAMS
    return pl.kernel(
        _make_gather_body(width),
        out_type=jax.ShapeDtypeStruct((n_edges, width), jnp.float32),
        mesh=_sc_mesh(),
        compiler_params=params,
        scratch_types=[
            pltpu.VMEM((per_w,), jnp.int32),
            pltpu.VMEM((per_w,), jnp.int32),
            pltpu.VMEM((CH, width), jnp.float32),
            pltpu.VMEM((CH, width), jnp.float32),
            pltpu.VMEM((CH, width), jnp.float32),
            pltpu.VMEM((CH, width), jnp.float32),
            pltpu.VMEM((CH, width), jnp.float32),
            pltpu.VMEM((CH, width), jnp.float32),
            pltpu.SemaphoreType.DMA,
            pltpu.SemaphoreType.DMA,
            pltpu.SemaphoreType.DMA,
            pltpu.SemaphoreType.DMA,
        ],
    )(ps, pd, src_f, dst_f)


def _edge_body(h_ref, r_ref, ea_ref, w1e_ref, b1_ref, w2_ref, wre_ref,
               bz_ref, gam_ref, bet_ref, out_ref, outp_ref):
    ea = ea_ref[...]
    h = (h_ref[...]
         + jnp.dot(ea, w1e_ref[...], preferred_element_type=jnp.float32)
         + b1_ref[...])
    y = h * (1.0 / (1.0 + jnp.exp(-h)))
    z = (jnp.dot(y, w2_ref[...], preferred_element_type=jnp.float32)
         + r_ref[...]
         + jnp.dot(ea, wre_ref[...], preferred_element_type=jnp.float32)
         + bz_ref[...])
    mu = jnp.mean(z, axis=1, keepdims=True)
    zc = z - mu
    var = jnp.mean(zc * zc, axis=1, keepdims=True)
    zn = zc * lax.rsqrt(var + _EPS) * gam_ref[...] + bet_ref[...]
    out_ref[...] = zn
    outp_ref[...] = jnp.concatenate(
        [zn, jnp.zeros((zn.shape[0], NODE_DIM - EDGE_DIM), jnp.float32)],
        axis=1)


def _edge_mlp_fused(h, r, ea, w1e, b1, w2, wre, bz, gam, bet):
    n_edges = h.shape[0]
    blk = 8000
    return pl.pallas_call(
        _edge_body,
        grid=(n_edges // blk,),
        in_specs=[
            pl.BlockSpec((blk, NODE_DIM), lambda i: (i, 0)),
            pl.BlockSpec((blk, EDGE_DIM), lambda i: (i, 0)),
            pl.BlockSpec((blk, EDGE_DIM), lambda i: (i, 0)),
            pl.BlockSpec((EDGE_DIM, NODE_DIM), lambda i: (0, 0)),
            pl.BlockSpec((1, NODE_DIM), lambda i: (0, 0)),
            pl.BlockSpec((NODE_DIM, EDGE_DIM), lambda i: (0, 0)),
            pl.BlockSpec((EDGE_DIM, EDGE_DIM), lambda i: (0, 0)),
            pl.BlockSpec((1, EDGE_DIM), lambda i: (0, 0)),
            pl.BlockSpec((1, EDGE_DIM), lambda i: (0, 0)),
            pl.BlockSpec((1, EDGE_DIM), lambda i: (0, 0)),
        ],
        out_specs=[
            pl.BlockSpec((blk, EDGE_DIM), lambda i: (i, 0)),
            pl.BlockSpec((blk, NODE_DIM), lambda i: (i, 0)),
        ],
        out_shape=[
            jax.ShapeDtypeStruct((n_edges, EDGE_DIM), jnp.float32),
            jax.ShapeDtypeStruct((n_edges, NODE_DIM), jnp.float32),
        ],
    )(h, r, ea, w1e, b1, w2, wre, bz, gam, bet)


def _scatter_body(ne_hbm, dst_hbm, out_hbm, agg_sh,
                  rows0, rows1, rows2, idx0, idx1, idx2, zb,
                  sl0, sl1, sl2, ss0, ss1, ss2):
    n_edges = dst_hbm.shape[0]
    per_w = n_edges // NW
    n_chunks = per_w // CH
    n_nodes = agg_sh.shape[0]
    zrows = zb.shape[0]
    n_zcopies = n_nodes // zrows
    c = lax.axis_index("c")
    s = lax.axis_index("s")
    wid = c * NS + s
    base = wid * per_w
    rows = (rows0, rows1, rows2)
    idx = (idx0, idx1, idx2)
    sl = (sl0, sl1, sl2)
    ss = (ss0, ss1, ss2)

    def zr(i, carry):
        for j in range(NODE_DIM // 16):
            zb[i, pl.ds(j * 16, 16)] = jnp.zeros((16,), jnp.float32)
        return carry

    lax.fori_loop(0, zrows, zr, 0)

    def zcopy(j, carry):
        k = s + NS * j

        @pl.when(k < n_zcopies)
        def _():
            pltpu.sync_copy(zb, agg_sh.at[pl.ds(k * zrows, zrows)])

        return carry

    lax.fori_loop(0, (n_zcopies + NS - 1) // NS, zcopy, 0)
    plsc.subcore_barrier()

    def fire(t, b):
        csl = pl.ds(base + t * CH, CH)
        pltpu.async_copy(ne_hbm.at[csl], rows[b], sl[b])
        pltpu.async_copy(dst_hbm.at[csl], idx[b], sl[b])

    def drain_scatter(b):
        pltpu.make_async_copy(rows[b], agg_sh.at[idx[b]], ss[b]).wait()

    def process(t, b, bn):
        pltpu.make_async_copy(ne_hbm.at[pl.ds(base, CH)], rows[b],
                              sl[b]).wait()
        pltpu.make_async_copy(dst_hbm.at[pl.ds(base, CH)], idx[b],
                              sl[b]).wait()
        pltpu.async_copy(rows[b], agg_sh.at[idx[b]], ss[b], add=True)

        @pl.when(t >= 1)
        def _():
            drain_scatter(bn)

        @pl.when(t + 2 < n_chunks)
        def _():
            fire(t + 2, bn)

    fire(0, 0)
    fire(1, 1)

    def step(tt, carry):
        for b in range(3):
            t = 3 * tt + b

            @pl.when(t < n_chunks)
            def _():
                process(t, b, (b + 2) % 3)

        return carry

    lax.fori_loop(0, (n_chunks + 2) // 3, step, 0)
    drain_scatter((n_chunks - 1) % 3)
    plsc.subcore_barrier()

    @pl.when(s == 0)
    def _writeout():
        pltpu.sync_copy(agg_sh, out_hbm.at[c])


def _sc_scatter_add(ne, dst_f, n_nodes, n_edges):
    return pl.kernel(
        _scatter_body,
        out_type=jax.ShapeDtypeStruct((NC, n_nodes, NODE_DIM), jnp.float32),
        mesh=_sc_mesh(),
        scratch_types=[
            pltpu.VMEM_SHARED((n_nodes, NODE_DIM), jnp.float32),
            pltpu.VMEM((CH, NODE_DIM), jnp.float32),
            pltpu.VMEM((CH, NODE_DIM), jnp.float32),
            pltpu.VMEM((CH, NODE_DIM), jnp.float32),
            pltpu.VMEM((CH,), jnp.int32),
            pltpu.VMEM((CH,), jnp.int32),
            pltpu.VMEM((CH,), jnp.int32),
            pltpu.VMEM((80, NODE_DIM), jnp.float32),
            pltpu.SemaphoreType.DMA,
            pltpu.SemaphoreType.DMA,
            pltpu.SemaphoreType.DMA,
            pltpu.SemaphoreType.DMA,
            pltpu.SemaphoreType.DMA,
            pltpu.SemaphoreType.DMA,
        ],
    )(ne, dst_f)


def _node_body(x_ref, ap_ref, w1x_ref, w1a_ref, b1_ref, w2_ref,
               wrx_ref, wra_ref, bz_ref, gam_ref, bet_ref, out_ref):
    xb = x_ref[...]
    agg = (ap_ref[0] + ap_ref[1])[:, :EDGE_DIM]
    h = (jnp.dot(xb, w1x_ref[...], preferred_element_type=jnp.float32)
         + jnp.dot(agg, w1a_ref[...], preferred_element_type=jnp.float32)
         + b1_ref[...])
    y = h * (1.0 / (1.0 + jnp.exp(-h)))
    z = (jnp.dot(y, w2_ref[...], preferred_element_type=jnp.float32)
         + jnp.dot(xb, wrx_ref[...], preferred_element_type=jnp.float32)
         + jnp.dot(agg, wra_ref[...], preferred_element_type=jnp.float32)
         + bz_ref[...])
    mu = jnp.mean(z, axis=1, keepdims=True)
    zc = z - mu
    var = jnp.mean(zc * zc, axis=1, keepdims=True)
    out_ref[...] = zc * lax.rsqrt(var + _EPS) * gam_ref[...] + bet_ref[...]


def _node_mlp_fused(x, aggp, w1x, w1a, b1, w2, wrx, wra, bz, gam, bet):
    n = x.shape[0]
    blk = 1000
    hd = w2.shape[0]
    return pl.pallas_call(
        _node_body,
        grid=(n // blk,),
        in_specs=[
            pl.BlockSpec((blk, NODE_DIM), lambda i: (i, 0)),
            pl.BlockSpec((NC, blk, NODE_DIM), lambda i: (0, i, 0)),
            pl.BlockSpec((NODE_DIM, hd), lambda i: (0, 0)),
            pl.BlockSpec((EDGE_DIM, hd), lambda i: (0, 0)),
            pl.BlockSpec((1, hd), lambda i: (0, 0)),
            pl.BlockSpec((hd, NODE_DIM), lambda i: (0, 0)),
            pl.BlockSpec((NODE_DIM, NODE_DIM), lambda i: (0, 0)),
            pl.BlockSpec((EDGE_DIM, NODE_DIM), lambda i: (0, 0)),
            pl.BlockSpec((1, NODE_DIM), lambda i: (0, 0)),
            pl.BlockSpec((1, NODE_DIM), lambda i: (0, 0)),
            pl.BlockSpec((1, NODE_DIM), lambda i: (0, 0)),
        ],
        out_specs=pl.BlockSpec((blk, NODE_DIM), lambda i: (i, 0)),
        out_shape=jax.ShapeDtypeStruct((n, NODE_DIM), jnp.float32),
    )(x, aggp, w1x, w1a, b1, w2, wrx, wra, bz, gam, bet)


def kernel(x, edge_attr, edge_index, edge_mlp, node_mlp):
    n_nodes = x.shape[0]
    n_edges = edge_attr.shape[0]

    src_f = edge_index[0].astype(jnp.int32)
    dst_f = edge_index[1].astype(jnp.int32)

    w1, wr = edge_mlp["W1"], edge_mlp["Wr"]
    ps, pd, rs, rd = _node_proj(
        x,
        w1[EDGE_DIM:EDGE_DIM + NODE_DIM],
        w1[EDGE_DIM + NODE_DIM:],
        wr[EDGE_DIM:EDGE_DIM + NODE_DIM],
        wr[EDGE_DIM + NODE_DIM:],
    )
    r = _sc_gather_add(rs, rd, src_f, dst_f, n_edges, EDGE_DIM)
    h = _sc_gather_add(ps, pd, src_f, dst_f, n_edges, NODE_DIM)

    new_edge, nep = _edge_mlp_fused(
        h, r, edge_attr,
        w1[:EDGE_DIM],
        edge_mlp["b1"].reshape(1, -1),
        edge_mlp["W2"],
        wr[:EDGE_DIM],
        (edge_mlp["b2"] + edge_mlp["br"]).reshape(1, -1),
        edge_mlp["gamma"].reshape(1, -1),
        edge_mlp["beta"].reshape(1, -1),
    )

    aggp = _sc_scatter_add(nep, dst_f, n_nodes, n_edges)

    nw1, nwr = node_mlp["W1"], node_mlp["Wr"]
    new_x = _node_mlp_fused(
        x, aggp,
        nw1[:NODE_DIM], nw1[NODE_DIM:],
        node_mlp["b1"].reshape(1, -1),
        node_mlp["W2"],
        nwr[:NODE_DIM], nwr[NODE_DIM:],
        (node_mlp["b2"] + node_mlp["br"]).reshape(1, -1),
        node_mlp["gamma"].reshape(1, -1),
        node_mlp["beta"].reshape(1, -1),
    )
    return new_x, new_edge


# speedup vs baseline: 1.5085x; 1.0016x over previous
import jax
import jax.numpy as jnp
from jax import lax
from jax.experimental import pallas as pl
from jax.experimental.pallas import tpu as pltpu
from jax.experimental.pallas import tpu_sc as plsc

NODE_DIM = 128
EDGE_DIM = 16

NC, NS = 2, 16
NW = NC * NS
CH = 80
SUBS = 5

_EPS = 1e-5

_SC_PARAMS = pltpu.CompilerParams(use_tc_tiling_on_sc=False)


def _sc_mesh():
    return plsc.VectorSubcoreMesh(core_axis_name="c", subcore_axis_name="s",
                                  num_cores=NC, num_subcores=NS)


def _proj_body(x_ref, w1s_ref, w1d_ref, wrs_ref, wrd_ref,
               ps_ref, pd_ref, rs_ref, rd_ref):
    xb = x_ref[...]
    ps_ref[...] = jnp.dot(xb, w1s_ref[...], preferred_element_type=jnp.float32)
    pd_ref[...] = jnp.dot(xb, w1d_ref[...], preferred_element_type=jnp.float32)
    rs_ref[...] = jnp.dot(xb, wrs_ref[...], preferred_element_type=jnp.float32)
    rd_ref[...] = jnp.dot(xb, wrd_ref[...], preferred_element_type=jnp.float32)


def _node_proj(x, w1s, w1d, wrs, wrd):
    n = x.shape[0]
    blk = 1000
    return pl.pallas_call(
        _proj_body,
        grid=(n // blk,),
        in_specs=[
            pl.BlockSpec((blk, NODE_DIM), lambda i: (i, 0)),
            pl.BlockSpec((NODE_DIM, NODE_DIM), lambda i: (0, 0)),
            pl.BlockSpec((NODE_DIM, NODE_DIM), lambda i: (0, 0)),
            pl.BlockSpec((NODE_DIM, EDGE_DIM), lambda i: (0, 0)),
            pl.BlockSpec((NODE_DIM, EDGE_DIM), lambda i: (0, 0)),
        ],
        out_specs=[
            pl.BlockSpec((blk, NODE_DIM), lambda i: (i, 0)),
            pl.BlockSpec((blk, NODE_DIM), lambda i: (i, 0)),
            pl.BlockSpec((blk, EDGE_DIM), lambda i: (i, 0)),
            pl.BlockSpec((blk, EDGE_DIM), lambda i: (i, 0)),
        ],
        out_shape=[
            jax.ShapeDtypeStruct((n, NODE_DIM), jnp.float32),
            jax.ShapeDtypeStruct((n, NODE_DIM), jnp.float32),
            jax.ShapeDtypeStruct((n, EDGE_DIM), jnp.float32),
            jax.ShapeDtypeStruct((n, EDGE_DIM), jnp.float32),
        ],
    )(x, w1s, w1d, wrs, wrd)


def _make_gather_body(width):
    def body(ps_hbm, pd_hbm, src_hbm, dst_hbm, h_hbm,
             idx_s, idx_d, bs0, bd0, bs1, bd1, bw0, bw1,
             sg0, sg1, sw0, sw1):
        per_w = idx_s.shape[0]
        n_chunks = per_w // CH
        c = lax.axis_index("c")
        s = lax.axis_index("s")
        wid = c * NS + s
        base = wid * per_w
        pltpu.sync_copy(src_hbm.at[pl.ds(base, per_w)], idx_s)
        pltpu.sync_copy(dst_hbm.at[pl.ds(base, per_w)], idx_d)
        bs = (bs0, bs1)
        bd = (bd0, bd1)
        bw = (bw0, bw1)
        sg = (sg0, sg1)
        sw = (sw0, sw1)

        def fire(t, b):
            tsl = pl.ds(t * CH, CH)
            pltpu.async_copy(ps_hbm.at[idx_s.at[tsl]], bs[b], sg[b])
            pltpu.async_copy(pd_hbm.at[idx_d.at[tsl]], bd[b], sg[b])

        def process(t, b):
            tsl = pl.ds(t * CH, CH)
            pltpu.make_async_copy(ps_hbm.at[idx_s.at[tsl]], bs[b],
                                  sg[b]).wait()
            pltpu.make_async_copy(pd_hbm.at[idx_d.at[tsl]], bd[b],
                                  sg[b]).wait()

            @pl.when(t >= 2)
            def _():
                pltpu.make_async_copy(bw[b], h_hbm.at[pl.ds(base, CH)],
                                      sw[b]).wait()

            def row(i, c2):
                for j in range(width // 16):
                    sl = pl.ds(j * 16, 16)
                    bw[b][i, sl] = bs[b][i, sl] + bd[b][i, sl]
                return c2

            lax.fori_loop(0, CH, row, 0)
            pltpu.async_copy(bw[b], h_hbm.at[pl.ds(base + t * CH, CH)], sw[b])

            @pl.when(t + 2 < n_chunks)
            def _():
                fire(t + 2, b)

        fire(0, 0)
        fire(1, 1)

        def step(tt, carry):
            for b in range(2):
                t = 2 * tt + b

                @pl.when(t < n_chunks)
                def _():
                    process(t, b)

            return carry

        lax.fori_loop(0, (n_chunks + 1) // 2, step, 0)
        pltpu.make_async_copy(bw0, h_hbm.at[pl.ds(base, CH)], sw0).wait()
        pltpu.make_async_copy(bw1, h_hbm.at[pl.ds(base, CH)], sw1).wait()

    return body


def _sc_gather_add(ps, pd, src_f, dst_f, n_edges, width):
    per_w = n_edges // NW
    params = None if width == NODE_DIM else _SC_PARAMS
    return pl.kernel(
        _make_gather_body(width),
        out_type=jax.ShapeDtypeStruct((n_edges, width), jnp.float32),
        mesh=_sc_mesh(),
        compiler_params=params,
        scratch_types=[
            pltpu.VMEM((per_w,), jnp.int32),
            pltpu.VMEM((per_w,), jnp.int32),
            pltpu.VMEM((CH, width), jnp.float32),
            pltpu.VMEM((CH, width), jnp.float32),
            pltpu.VMEM((CH, width), jnp.float32),
            pltpu.VMEM((CH, width), jnp.float32),
            pltpu.VMEM((CH, width), jnp.float32),
            pltpu.VMEM((CH, width), jnp.float32),
            pltpu.SemaphoreType.DMA,
            pltpu.SemaphoreType.DMA,
            pltpu.SemaphoreType.DMA,
            pltpu.SemaphoreType.DMA,
        ],
    )(ps, pd, src_f, dst_f)


def _edge_body(h_ref, r_ref, ea_ref, w1e_ref, b1_ref, w2_ref, wre_ref,
               bz_ref, gam_ref, bet_ref, out_ref, outp_ref):
    ea = ea_ref[...]
    h = (h_ref[...]
         + jnp.dot(ea, w1e_ref[...], preferred_element_type=jnp.float32)
         + b1_ref[...])
    y = h * (1.0 / (1.0 + jnp.exp(-h)))
    z = (jnp.dot(y, w2_ref[...], preferred_element_type=jnp.float32)
         + r_ref[...]
         + jnp.dot(ea, wre_ref[...], preferred_element_type=jnp.float32)
         + bz_ref[...])
    mu = jnp.mean(z, axis=1, keepdims=True)
    zc = z - mu
    var = jnp.mean(zc * zc, axis=1, keepdims=True)
    zn = zc * lax.rsqrt(var + _EPS) * gam_ref[...] + bet_ref[...]
    out_ref[...] = zn
    outp_ref[...] = jnp.concatenate(
        [zn, jnp.zeros((zn.shape[0], NODE_DIM - EDGE_DIM), jnp.float32)],
        axis=1)


def _edge_mlp_fused(h, r, ea, w1e, b1, w2, wre, bz, gam, bet):
    n_edges = h.shape[0]
    blk = 8000
    return pl.pallas_call(
        _edge_body,
        grid=(n_edges // blk,),
        in_specs=[
            pl.BlockSpec((blk, NODE_DIM), lambda i: (i, 0)),
            pl.BlockSpec((blk, EDGE_DIM), lambda i: (i, 0)),
            pl.BlockSpec((blk, EDGE_DIM), lambda i: (i, 0)),
            pl.BlockSpec((EDGE_DIM, NODE_DIM), lambda i: (0, 0)),
            pl.BlockSpec((1, NODE_DIM), lambda i: (0, 0)),
            pl.BlockSpec((NODE_DIM, EDGE_DIM), lambda i: (0, 0)),
            pl.BlockSpec((EDGE_DIM, EDGE_DIM), lambda i: (0, 0)),
            pl.BlockSpec((1, EDGE_DIM), lambda i: (0, 0)),
            pl.BlockSpec((1, EDGE_DIM), lambda i: (0, 0)),
            pl.BlockSpec((1, EDGE_DIM), lambda i: (0, 0)),
        ],
        out_specs=[
            pl.BlockSpec((blk, EDGE_DIM), lambda i: (i, 0)),
            pl.BlockSpec((blk, NODE_DIM), lambda i: (i, 0)),
        ],
        out_shape=[
            jax.ShapeDtypeStruct((n_edges, EDGE_DIM), jnp.float32),
            jax.ShapeDtypeStruct((n_edges, NODE_DIM), jnp.float32),
        ],
    )(h, r, ea, w1e, b1, w2, wre, bz, gam, bet)


def _scatter_body(ne_hbm, dst_hbm, out_hbm, agg_sh,
                  rows0, rows1, rows2, idx0, idx1, idx2, zb,
                  sl0, sl1, sl2):
    n_edges = dst_hbm.shape[0]
    per_w = n_edges // NW
    n_chunks = per_w // CH
    n_nodes = agg_sh.shape[0]
    zrows = zb.shape[0]
    n_zcopies = n_nodes // zrows
    c = lax.axis_index("c")
    s = lax.axis_index("s")
    wid = c * NS + s
    base = wid * per_w
    rows = (rows0, rows1, rows2)
    idx = (idx0, idx1, idx2)
    sl = (sl0, sl1, sl2)

    def zr(i, carry):
        for j in range(NODE_DIM // 16):
            zb[i, pl.ds(j * 16, 16)] = jnp.zeros((16,), jnp.float32)
        return carry

    lax.fori_loop(0, zrows, zr, 0)

    def zcopy(j, carry):
        k = s + NS * j

        @pl.when(k < n_zcopies)
        def _():
            pltpu.sync_copy(zb, agg_sh.at[pl.ds(k * zrows, zrows)])

        return carry

    lax.fori_loop(0, (n_zcopies + NS - 1) // NS, zcopy, 0)
    plsc.subcore_barrier()

    def fire(t, b):
        csl = pl.ds(base + t * CH, CH)
        pltpu.async_copy(ne_hbm.at[csl], rows[b], sl[b])
        pltpu.async_copy(dst_hbm.at[csl], idx[b], sl[b])

    def process(t, b, bn):
        pltpu.make_async_copy(ne_hbm.at[pl.ds(base, CH)], rows[b],
                              sl[b]).wait()
        pltpu.make_async_copy(dst_hbm.at[pl.ds(base, CH)], idx[b],
                              sl[b]).wait()

        @pl.when(t + 2 < n_chunks)
        def _():
            fire(t + 2, bn)

        pltpu.sync_copy(rows[b], agg_sh.at[idx[b]], add=True)

    fire(0, 0)
    fire(1, 1)

    def step(tt, carry):
        for b in range(3):
            t = 3 * tt + b

            @pl.when(t < n_chunks)
            def _():
                process(t, b, (b + 2) % 3)

        return carry

    lax.fori_loop(0, (n_chunks + 2) // 3, step, 0)
    plsc.subcore_barrier()

    @pl.when(s == 0)
    def _writeout():
        pltpu.sync_copy(agg_sh, out_hbm.at[c])


def _sc_scatter_add(ne, dst_f, n_nodes, n_edges):
    return pl.kernel(
        _scatter_body,
        out_type=jax.ShapeDtypeStruct((NC, n_nodes, NODE_DIM), jnp.float32),
        mesh=_sc_mesh(),
        scratch_types=[
            pltpu.VMEM_SHARED((n_nodes, NODE_DIM), jnp.float32),
            pltpu.VMEM((CH, NODE_DIM), jnp.float32),
            pltpu.VMEM((CH, NODE_DIM), jnp.float32),
            pltpu.VMEM((CH, NODE_DIM), jnp.float32),
            pltpu.VMEM((CH,), jnp.int32),
            pltpu.VMEM((CH,), jnp.int32),
            pltpu.VMEM((CH,), jnp.int32),
            pltpu.VMEM((80, NODE_DIM), jnp.float32),
            pltpu.SemaphoreType.DMA,
            pltpu.SemaphoreType.DMA,
            pltpu.SemaphoreType.DMA,
        ],
    )(ne, dst_f)


def _node_body(x_ref, ap_ref, w1x_ref, w1a_ref, b1_ref, w2_ref,
               wrx_ref, wra_ref, bz_ref, gam_ref, bet_ref, out_ref):
    xb = x_ref[...]
    agg = (ap_ref[0] + ap_ref[1])[:, :EDGE_DIM]
    h = (jnp.dot(xb, w1x_ref[...], preferred_element_type=jnp.float32)
         + jnp.dot(agg, w1a_ref[...], preferred_element_type=jnp.float32)
         + b1_ref[...])
    y = h * (1.0 / (1.0 + jnp.exp(-h)))
    z = (jnp.dot(y, w2_ref[...], preferred_element_type=jnp.float32)
         + jnp.dot(xb, wrx_ref[...], preferred_element_type=jnp.float32)
         + jnp.dot(agg, wra_ref[...], preferred_element_type=jnp.float32)
         + bz_ref[...])
    mu = jnp.mean(z, axis=1, keepdims=True)
    zc = z - mu
    var = jnp.mean(zc * zc, axis=1, keepdims=True)
    out_ref[...] = zc * lax.rsqrt(var + _EPS) * gam_ref[...] + bet_ref[...]


def _node_mlp_fused(x, aggp, w1x, w1a, b1, w2, wrx, wra, bz, gam, bet):
    n = x.shape[0]
    blk = 1000
    hd = w2.shape[0]
    return pl.pallas_call(
        _node_body,
        grid=(n // blk,),
        in_specs=[
            pl.BlockSpec((blk, NODE_DIM), lambda i: (i, 0)),
            pl.BlockSpec((NC, blk, NODE_DIM), lambda i: (0, i, 0)),
            pl.BlockSpec((NODE_DIM, hd), lambda i: (0, 0)),
            pl.BlockSpec((EDGE_DIM, hd), lambda i: (0, 0)),
            pl.BlockSpec((1, hd), lambda i: (0, 0)),
            pl.BlockSpec((hd, NODE_DIM), lambda i: (0, 0)),
            pl.BlockSpec((NODE_DIM, NODE_DIM), lambda i: (0, 0)),
            pl.BlockSpec((EDGE_DIM, NODE_DIM), lambda i: (0, 0)),
            pl.BlockSpec((1, NODE_DIM), lambda i: (0, 0)),
            pl.BlockSpec((1, NODE_DIM), lambda i: (0, 0)),
            pl.BlockSpec((1, NODE_DIM), lambda i: (0, 0)),
        ],
        out_specs=pl.BlockSpec((blk, NODE_DIM), lambda i: (i, 0)),
        out_shape=jax.ShapeDtypeStruct((n, NODE_DIM), jnp.float32),
    )(x, aggp, w1x, w1a, b1, w2, wrx, wra, bz, gam, bet)


def kernel(x, edge_attr, edge_index, edge_mlp, node_mlp):
    n_nodes = x.shape[0]
    n_edges = edge_attr.shape[0]

    src_f = edge_index[0].astype(jnp.int32)
    dst_f = edge_index[1].astype(jnp.int32)

    w1, wr = edge_mlp["W1"], edge_mlp["Wr"]
    ps, pd, rs, rd = _node_proj(
        x,
        w1[EDGE_DIM:EDGE_DIM + NODE_DIM],
        w1[EDGE_DIM + NODE_DIM:],
        wr[EDGE_DIM:EDGE_DIM + NODE_DIM],
        wr[EDGE_DIM + NODE_DIM:],
    )
    r = _sc_gather_add(rs, rd, src_f, dst_f, n_edges, EDGE_DIM)
    h = _sc_gather_add(ps, pd, src_f, dst_f, n_edges, NODE_DIM)

    new_edge, nep = _edge_mlp_fused(
        h, r, edge_attr,
        w1[:EDGE_DIM],
        edge_mlp["b1"].reshape(1, -1),
        edge_mlp["W2"],
        wr[:EDGE_DIM],
        (edge_mlp["b2"] + edge_mlp["br"]).reshape(1, -1),
        edge_mlp["gamma"].reshape(1, -1),
        edge_mlp["beta"].reshape(1, -1),
    )

    aggp = _sc_scatter_add(nep, dst_f, n_nodes, n_edges)

    nw1, nwr = node_mlp["W1"], node_mlp["Wr"]
    new_x = _node_mlp_fused(
        x, aggp,
        nw1[:NODE_DIM], nw1[NODE_DIM:],
        node_mlp["b1"].reshape(1, -1),
        node_mlp["W2"],
        nwr[:NODE_DIM], nwr[NODE_DIM:],
        (node_mlp["b2"] + node_mlp["br"]).reshape(1, -1),
        node_mlp["gamma"].reshape(1, -1),
        node_mlp["beta"].reshape(1, -1),
    )
    return new_x, new_edge
